# Initial kernel scaffold; baseline (speedup 1.0000x reference)
import jax
import jax.numpy as jnp
from jax.experimental import pallas as pl


def kernel(x, edge_index, edge_attr, W1, as1, ad1, ae1, le1, b1, W2, as2, ad2, ae2, le2, b2, W3, as3, ad3, ae3, le3, b3, W4, as4, ad4, ae4, le4, b4, Wv1, bv1, Wv2, bv2):
    raise NotImplementedError("write your pallas kernel here")


# speedup vs baseline: 1.0000x; 1.0000x over previous
import jax
import jax.numpy as jnp
from jax.experimental import pallas as pl


def _gat(x, src, dst, ea, W, a_s, a_d, a_e, We, b, n):
    h = x @ W
    alpha = (h * a_s).sum(-1)[src] + (h * a_d).sum(-1)[dst] + ((ea @ We) * a_e).sum(-1)
    alpha = jax.nn.leaky_relu(alpha, 0.2)
    m = jax.lax.stop_gradient(jax.ops.segment_max(alpha, dst, num_segments=n))
    ex = jnp.exp(alpha - m[dst])
    den = jax.ops.segment_sum(ex, dst, num_segments=n)
    coef = ex / (den[dst] + 1e-16)
    return jax.ops.segment_sum(coef[:, None] * h[src], dst, num_segments=n) + b


def _gat_sl(x, edge_index, ea, W, a_s, a_d, a_e, We, b):
    n = x.shape[0]
    src, dst = edge_index[0], edge_index[1]
    s = jax.ops.segment_sum(ea, dst, num_segments=n)
    c = jax.ops.segment_sum(jnp.ones((ea.shape[0],), ea.dtype), dst, num_segments=n)
    loop = s / jnp.maximum(c, 1.0)[:, None]
    ar = jnp.arange(n, dtype=src.dtype)
    src2 = jnp.concatenate([src, ar])
    dst2 = jnp.concatenate([dst, ar])
    ea2 = jnp.concatenate([ea, loop], axis=0)
    return _gat(x, src2, dst2, ea2, W, a_s, a_d, a_e, We, b, n)


def _mlp_kernel(v_ref, w1_ref, b1_ref, w2_ref, b2_ref, o_ref):
    v = v_ref[...]
    vx = jnp.maximum(v @ w1_ref[...] + b1_ref[...], 0.0)
    o_ref[...] = vx @ w2_ref[...] + b2_ref[...]


def kernel(x, edge_index, edge_attr, W1, as1, ad1, ae1, le1, b1, W2, as2, ad2, ae2, le2, b2, W3, as3, ad3, ae3, le3, b3, W4, as4, ad4, ae4, le4, b4, Wv1, bv1, Wv2, bv2):
    OUT_SIZE = 10000
    H = 16
    ew = edge_attr[:, :2]
    xa = jax.nn.relu(_gat_sl(x, edge_index, ew, W1, as1, ad1, ae1, le1, b1))
    xa = jax.nn.relu(_gat_sl(jnp.concatenate([xa, x], axis=1), edge_index, ew, W2, as2, ad2, ae2, le2, b2))
    xa = jax.nn.relu(_gat_sl(jnp.concatenate([xa, x], axis=1), edge_index, ew, W3, as3, ad3, ae3, le3, b3))
    px = jax.nn.relu(_gat_sl(jnp.concatenate([xa, x], axis=1), edge_index, ew, W4, as4, ad4, ae4, le4, b4))
    px = px.reshape(-1, min(OUT_SIZE, px.shape[0]))
    xa3 = xa.reshape(-1, min(OUT_SIZE, xa.shape[0]), H)
    v = xa3.mean(axis=1)
    vx = pl.pallas_call(
        _mlp_kernel,
        out_shape=jax.ShapeDtypeStruct((v.shape[0], 1), jnp.float32),
    )(v, Wv1, bv1[None, :], Wv2, bv2[None, :])
    return (px, vx)


# speedup vs baseline: 41.5054x; 41.5051x over previous
import functools

import jax
import jax.numpy as jnp
from jax import lax
from jax.experimental import pallas as pl
from jax.experimental.pallas import tpu as pltpu
from jax.experimental.pallas import tpu_sc as plsc

N = 10000
NPAD = 10112
E = 320000
NW = 32
RPT = 80
EPW = RPT * 128
EPAD = NW * EPW
ER = EPAD // 128
H = 16

_mesh = plsc.VectorSubcoreMesh(core_axis_name="c", subcore_axis_name="s")
_sc_params = pltpu.CompilerParams(needs_layout_passes=False, use_tc_tiling_on_sc=False)
f32 = jnp.float32
i32 = jnp.int32


def _wid_base():
    c = lax.axis_index("c")
    s = lax.axis_index("s")
    return c, s, (c * 16 + s) * RPT


@functools.partial(
    pl.kernel,
    out_type=(jax.ShapeDtypeStruct((2 * NPAD,), f32),
              jax.ShapeDtypeStruct((2 * NPAD,), f32),
              jax.ShapeDtypeStruct((2 * NPAD,), f32)),
    mesh=_mesh,
    compiler_params=_sc_params,
    scratch_types=[
        pltpu.VMEM((RPT, 128), i32),
        pltpu.VMEM((RPT, 128), f32),
        pltpu.VMEM((RPT, 128), f32),
        pltpu.VMEM((RPT, 128), f32),
        pltpu.VMEM_SHARED((NPAD,), f32),
        pltpu.VMEM_SHARED((NPAD,), f32),
        pltpu.VMEM_SHARED((NPAD,), f32),
    ],
)
def _sc_pass0(dst_h, ew0_h, ew1_h, ones_h, zeros_h, outA_h, outB_h, outC_h,
              dstb, e0b, e1b, oneb, accA, accB, accC):
    c, s, base = _wid_base()

    @pl.when(s == 0)
    def _():
        pltpu.sync_copy(zeros_h, accA)
        pltpu.sync_copy(zeros_h, accB)
        pltpu.sync_copy(zeros_h, accC)

    pltpu.sync_copy(dst_h.at[pl.ds(base, RPT)], dstb)
    pltpu.sync_copy(ew0_h.at[pl.ds(base, RPT)], e0b)
    pltpu.sync_copy(ew1_h.at[pl.ds(base, RPT)], e1b)
    pltpu.sync_copy(ones_h, oneb)

    plsc.subcore_barrier()

    def srow(j, carry):
        pltpu.sync_copy(e0b.at[j], accA.at[dstb.at[j]], add=True)
        pltpu.sync_copy(e1b.at[j], accB.at[dstb.at[j]], add=True)
        pltpu.sync_copy(oneb.at[j], accC.at[dstb.at[j]], add=True)
        return carry

    lax.fori_loop(0, RPT, srow, 0)
    plsc.subcore_barrier()

    @pl.when(s == 0)
    def _():
        pltpu.sync_copy(accA, outA_h.at[pl.ds(c * NPAD, NPAD)])
        pltpu.sync_copy(accB, outB_h.at[pl.ds(c * NPAD, NPAD)])
        pltpu.sync_copy(accC, outC_h.at[pl.ds(c * NPAD, NPAD)])


@functools.partial(
    pl.kernel,
    out_type=(jax.ShapeDtypeStruct((ER, 128), f32),
              jax.ShapeDtypeStruct((2 * NPAD,), f32)),
    mesh=_mesh,
    compiler_params=_sc_params,
    scratch_types=[
        pltpu.VMEM((RPT, 128), i32),
        pltpu.VMEM((RPT, 128), i32),
        pltpu.VMEM((RPT, 128), f32),
        pltpu.VMEM((RPT, 128), f32),
        pltpu.VMEM((RPT, 128), f32),
        pltpu.VMEM((NPAD,), f32),
        pltpu.VMEM((NPAD,), f32),
        pltpu.VMEM((NPAD,), f32),
        pltpu.VMEM((2, 16), f32),
        pltpu.VMEM_SHARED((NPAD,), f32),
    ],
)
def _sc_passA(src_h, dst_h, ew0_h, ew1_h, ssrc_h, sdst_h, msl_h, cb_h, zeros_h,
              ex_h, denp_h,
              srcb, dstb, e0b, e1b, exb, ssrcT, sdstT, mslT, cbb, den_sh):
    c, s, base = _wid_base()

    @pl.when(s == 0)
    def _():
        pltpu.sync_copy(zeros_h, den_sh)

    pltpu.sync_copy(src_h.at[pl.ds(base, RPT)], srcb)
    pltpu.sync_copy(dst_h.at[pl.ds(base, RPT)], dstb)
    pltpu.sync_copy(ew0_h.at[pl.ds(base, RPT)], e0b)
    pltpu.sync_copy(ew1_h.at[pl.ds(base, RPT)], e1b)
    pltpu.sync_copy(ssrc_h, ssrcT)
    pltpu.sync_copy(sdst_h, sdstT)
    pltpu.sync_copy(msl_h, mslT)
    pltpu.sync_copy(cb_h, cbb)

    c0 = cbb[0, :]
    c1 = cbb[1, :]

    def vrow(j, carry):
        def vcol(k, carry2):
            sl = pl.ds(k * 16, 16)
            s16 = srcb[j, sl]
            d16 = dstb[j, sl]
            e0 = e0b[j, sl]
            e1 = e1b[j, sl]
            g1 = plsc.load_gather(ssrcT, [s16])
            g2 = plsc.load_gather(sdstT, [d16])
            g3 = plsc.load_gather(mslT, [d16])
            al = g1 + g2 + e0 * c0 + e1 * c1
            al = jnp.maximum(al, al * 0.2)
            t = jnp.minimum(al - g3, 60.0)
            exb[j, sl] = jnp.exp(t)
            return carry2
        lax.fori_loop(0, 8, vcol, 0)
        return carry

    lax.fori_loop(0, RPT, vrow, 0)
    plsc.subcore_barrier()

    def srow(j, carry):
        pltpu.sync_copy(exb.at[j], den_sh.at[dstb.at[j]], add=True)
        return carry

    lax.fori_loop(0, RPT, srow, 0)
    pltpu.sync_copy(exb, ex_h.at[pl.ds(base, RPT)])
    plsc.subcore_barrier()

    @pl.when(s == 0)
    def _():
        pltpu.sync_copy(den_sh, denp_h.at[pl.ds(c * NPAD, NPAD)])


@functools.partial(
    pl.kernel,
    out_type=jax.ShapeDtypeStruct((2, NPAD, H), f32),
    mesh=_mesh,
    compiler_params=_sc_params,
    scratch_types=[
        pltpu.VMEM((RPT, 128), i32),
        pltpu.VMEM((RPT, 128), i32),
        pltpu.VMEM((RPT, 128), f32),
        pltpu.VMEM((RPT, 128), f32),
        pltpu.VMEM((NPAD,), f32),
        pltpu.VMEM((NPAD,), f32),
        pltpu.VMEM((128, H), f32),
        pltpu.VMEM_SHARED((NPAD, H), f32),
        pltpu.SemaphoreType.DMA,
    ],
)
def _sc_passB(src_h, dst_h, ex_h, denp_h, h_h, zeros2_h,
              outp_h,
              srcb, dstb, exb, cfb, denT, denT2, rows, out_sh, gsem):
    c, s, base = _wid_base()

    @pl.when(s == 0)
    def _():
        pltpu.sync_copy(zeros2_h, out_sh)

    pltpu.sync_copy(src_h.at[pl.ds(base, RPT)], srcb)
    pltpu.sync_copy(dst_h.at[pl.ds(base, RPT)], dstb)
    pltpu.sync_copy(ex_h.at[pl.ds(base, RPT)], exb)
    pltpu.sync_copy(denp_h.at[pl.ds(0, NPAD)], denT)
    pltpu.sync_copy(denp_h.at[pl.ds(NPAD, NPAD)], denT2)

    def dmerge(i, carry):
        sl = pl.ds(i * 16, 16)
        denT[sl] = denT[sl] + denT2[sl] + 1.0
        return carry

    lax.fori_loop(0, NPAD // 16, dmerge, 0)

    def vrow(j, carry):
        def vcol(k, carry2):
            sl = pl.ds(k * 16, 16)
            d16 = dstb[j, sl]
            dg = plsc.load_gather(denT, [d16])
            cfb[j, sl] = exb[j, sl] / (dg + 1e-16)
            return carry2
        lax.fori_loop(0, 8, vcol, 0)
        return carry

    lax.fori_loop(0, RPT, vrow, 0)
    plsc.subcore_barrier()

    zi = jnp.zeros((16,), i32)

    def rowphase(j, carry):
        pltpu.async_copy(h_h.at[srcb.at[j]], rows, gsem).wait()
        jsplat = zi + j

        def sbody(i, cidx):
            g = plsc.load_gather(cfb, [jsplat, cidx])
            rows[i, :] = rows[i, :] * g
            return cidx + 1

        lax.fori_loop(0, 128, sbody, zi)
        pltpu.sync_copy(rows, out_sh.at[dstb.at[j]], add=True)
        return carry

    lax.fori_loop(0, RPT, rowphase, 0)
    plsc.subcore_barrier()

    @pl.when(s == 0)
    def _():
        pltpu.sync_copy(out_sh, outp_h.at[c])


@functools.partial(
    pl.kernel,
    out_type=jax.ShapeDtypeStruct((2 * NPAD,), f32),
    mesh=_mesh,
    compiler_params=_sc_params,
    scratch_types=[
        pltpu.VMEM((RPT, 128), i32),
        pltpu.VMEM((RPT, 128), i32),
        pltpu.VMEM((RPT, 128), f32),
        pltpu.VMEM((RPT, 128), f32),
        pltpu.VMEM((NPAD,), f32),
        pltpu.VMEM((NPAD,), f32),
        pltpu.VMEM((NPAD,), f32),
        pltpu.VMEM_SHARED((NPAD,), f32),
    ],
)
def _sc_passB1(src_h, dst_h, ex_h, denp_h, h1_h, zeros_h,
               outp_h,
               srcb, dstb, exb, vb, denT, denT2, h1T, out_sh):
    c, s, base = _wid_base()

    @pl.when(s == 0)
    def _():
        pltpu.sync_copy(zeros_h, out_sh)

    pltpu.sync_copy(src_h.at[pl.ds(base, RPT)], srcb)
    pltpu.sync_copy(dst_h.at[pl.ds(base, RPT)], dstb)
    pltpu.sync_copy(ex_h.at[pl.ds(base, RPT)], exb)
    pltpu.sync_copy(denp_h.at[pl.ds(0, NPAD)], denT)
    pltpu.sync_copy(denp_h.at[pl.ds(NPAD, NPAD)], denT2)
    pltpu.sync_copy(h1_h, h1T)

    def dmerge(i, carry):
        sl = pl.ds(i * 16, 16)
        denT[sl] = denT[sl] + denT2[sl] + 1.0
        return carry

    lax.fori_loop(0, NPAD // 16, dmerge, 0)

    def vrow(j, carry):
        def vcol(k, carry2):
            sl = pl.ds(k * 16, 16)
            s16 = srcb[j, sl]
            d16 = dstb[j, sl]
            dg = plsc.load_gather(denT, [d16])
            hg = plsc.load_gather(h1T, [s16])
            vb[j, sl] = exb[j, sl] / (dg + 1e-16) * hg
            return carry2
        lax.fori_loop(0, 8, vcol, 0)
        return carry

    lax.fori_loop(0, RPT, vrow, 0)
    plsc.subcore_barrier()

    def srow(j, carry):
        pltpu.sync_copy(vb.at[j], out_sh.at[dstb.at[j]], add=True)
        return carry

    lax.fori_loop(0, RPT, srow, 0)
    plsc.subcore_barrier()

    @pl.when(s == 0)
    def _():
        pltpu.sync_copy(out_sh, outp_h.at[pl.ds(c * NPAD, NPAD)])


def _attn_scalars(h, a_s, a_d, le, ae, loop0, loop1):
    ssrc = jnp.sum(h * a_s, axis=-1)
    sdst = jnp.sum(h * a_d, axis=-1)
    cvec = le @ ae
    t = ssrc + sdst + loop0 * cvec[0] + loop1 * cvec[1]
    msl = jnp.maximum(t, t * 0.2)
    cb = jnp.broadcast_to(cvec[:, None], (2, 16))
    return ssrc, sdst, msl, cb


def _tc_prep1(xp_ref, W1_ref, as1_ref, ad1_ref, le1_ref, ae1_ref,
              pA_ref, pB_ref, pC_ref,
              h_ref, ssrc_ref, sdst_ref, msl_ref, cb_ref, l0_ref, l1_ref):
    xp = xp_ref[...]
    h = jax.lax.dot(xp, W1_ref[...], preferred_element_type=f32)
    pA = pA_ref[...]
    pB = pB_ref[...]
    pC = pC_ref[...]
    cnt = jnp.maximum(pC[:NPAD] + pC[NPAD:], 1.0)
    loop0 = (pA[:NPAD] + pA[NPAD:]) / cnt
    loop1 = (pB[:NPAD] + pB[NPAD:]) / cnt
    ssrc, sdst, msl, cb = _attn_scalars(
        h, as1_ref[...], ad1_ref[...], le1_ref[...], ae1_ref[...], loop0, loop1)
    h_ref[...] = h
    ssrc_ref[...] = ssrc
    sdst_ref[...] = sdst
    msl_ref[...] = msl
    cb_ref[...] = cb
    l0_ref[...] = loop0
    l1_ref[...] = loop1


def _tc_fin(outp_ref, denp_ref, h_ref, b_ref, xa_ref):
    denp = denp_ref[...]
    den = denp[:NPAD] + denp[NPAD:] + 1.0
    xa = outp_ref[0] + outp_ref[1] + h_ref[...] / den[:, None]
    xa = jnp.maximum(xa + b_ref[...], 0.0)
    rmask = lax.broadcasted_iota(i32, (NPAD, H), 0) < N
    xa_ref[...] = jnp.where(rmask, xa, 0.0)


def _tc_prepn(xa_ref, xp_ref, W_ref, as_ref, ad_ref, le_ref, ae_ref,
              l0_ref, l1_ref,
              hn_ref, ssrc_ref, sdst_ref, msl_ref, cb_ref):
    hn = (jax.lax.dot(xa_ref[...], W_ref[0:H, :], preferred_element_type=f32)
          + jax.lax.dot(xp_ref[...], W_ref[H:, :], preferred_element_type=f32))
    ssrc, sdst, msl, cb = _attn_scalars(
        hn, as_ref[...], ad_ref[...], le_ref[...], ae_ref[...],
        l0_ref[...], l1_ref[...])
    hn_ref[...] = hn
    ssrc_ref[...] = ssrc
    sdst_ref[...] = sdst
    msl_ref[...] = msl
    cb_ref[...] = cb


def _tc_prep4(xa_ref, xp_ref, W_ref, as_ref, ad_ref, le_ref, ae_ref,
              l0_ref, l1_ref,
              hn_ref, ssrc_ref, sdst_ref, msl_ref, cb_ref):
    v1 = W_ref[0:H, 0]
    v2 = W_ref[H:, 0]
    hn = jnp.sum(xa_ref[...] * v1, axis=-1) + jnp.sum(xp_ref[...] * v2, axis=-1)
    ssrc = hn * as_ref[0]
    sdst = hn * ad_ref[0]
    cvec = le_ref[:, 0] * ae_ref[0]
    t = ssrc + sdst + l0_ref[...] * cvec[0] + l1_ref[...] * cvec[1]
    msl = jnp.maximum(t, t * 0.2)
    hn_ref[...] = hn
    ssrc_ref[...] = ssrc
    sdst_ref[...] = sdst
    msl_ref[...] = msl
    cb_ref[...] = jnp.broadcast_to(cvec[:, None], (2, 16))


def _tc_head(outp4_ref, denp4_ref, h4f_ref, b4_ref, xa3_ref,
             Wv1_ref, bv1_ref, Wv2_ref, bv2_ref,
             px_ref, vx_ref):
    denp4 = denp4_ref[...]
    outp4 = outp4_ref[...]
    den = denp4[:NPAD] + denp4[NPAD:] + 1.0
    p = outp4[:NPAD] + outp4[NPAD:] + h4f_ref[...] / den + b4_ref[0]
    p = jnp.maximum(p, 0.0)
    px_ref[...] = p[:N][None, :]
    v = jnp.sum(xa3_ref[...], axis=0) / float(N)
    vx = jnp.maximum(v @ Wv1_ref[...] + bv1_ref[...], 0.0)
    vx_ref[...] = (vx @ Wv2_ref[...] + bv2_ref[...])[None, :]


def _pc(fn, out_shape):
    return pl.pallas_call(fn, out_shape=out_shape)


def kernel(x, edge_index, edge_attr, W1, as1, ad1, ae1, le1, b1,
           W2, as2, ad2, ae2, le2, b2, W3, as3, ad3, ae3, le3, b3,
           W4, as4, ad4, ae4, le4, b4, Wv1, bv1, Wv2, bv2):
    src = edge_index[0]
    dst = edge_index[1]
    pad_e = EPAD - E
    padi = jnp.full((pad_e,), N, i32)
    src2 = jnp.concatenate([src, padi]).reshape(ER, 128)
    dst2 = jnp.concatenate([dst, padi]).reshape(ER, 128)
    padf = jnp.zeros((pad_e,), f32)
    ew0 = jnp.concatenate([edge_attr[:, 0], padf]).reshape(ER, 128)
    ew1 = jnp.concatenate([edge_attr[:, 1], padf]).reshape(ER, 128)
    xp = jnp.pad(x, ((0, NPAD - N), (0, 0)))
    zN = jnp.zeros((NPAD,), f32)
    zNH = jnp.zeros((NPAD, H), f32)
    onesR = jnp.ones((RPT, 128), f32)

    pA, pB, pC = _sc_pass0(dst2, ew0, ew1, onesR, zN)

    sds = jax.ShapeDtypeStruct
    h1, ssrc, sdst, msl, cb, loop0, loop1 = _pc(
        _tc_prep1,
        (sds((NPAD, H), f32), sds((NPAD,), f32), sds((NPAD,), f32),
         sds((NPAD,), f32), sds((2, 16), f32), sds((NPAD,), f32),
         sds((NPAD,), f32)),
    )(xp, W1, as1, ad1, le1, ae1, pA, pB, pC)

    layers = [
        (b1, W2, as2, ad2, le2, ae2),
        (b2, W3, as3, ad3, le3, ae3),
        (b3, W4, as4, ad4, le4, ae4),
    ]

    h_cur = h1
    xa3 = None
    for li, (b_l, Wn, asn, adn, len_, aen) in enumerate(layers):
        ex, denp = _sc_passA(src2, dst2, ew0, ew1, ssrc, sdst, msl, cb, zN)
        outp = _sc_passB(src2, dst2, ex, denp, h_cur, zNH)
        xa = _pc(_tc_fin, sds((NPAD, H), f32))(outp, denp, h_cur, b_l)
        if li < 2:
            h_cur, ssrc, sdst, msl, cb = _pc(
                _tc_prepn,
                (sds((NPAD, H), f32), sds((NPAD,), f32), sds((NPAD,), f32),
                 sds((NPAD,), f32), sds((2, 16), f32)),
            )(xa, xp, Wn, asn, adn, len_, aen, loop0, loop1)
        else:
            h4flat, ssrc, sdst, msl, cb = _pc(
                _tc_prep4,
                (sds((NPAD,), f32), sds((NPAD,), f32), sds((NPAD,), f32),
                 sds((NPAD,), f32), sds((2, 16), f32)),
            )(xa, xp, Wn, asn, adn, len_, aen, loop0, loop1)
            xa3 = xa

    ex, denp = _sc_passA(src2, dst2, ew0, ew1, ssrc, sdst, msl, cb, zN)
    outp4 = _sc_passB1(src2, dst2, ex, denp, h4flat, zN)

    px, vx = _pc(
        _tc_head,
        (sds((1, N), f32), sds((1, 1), f32)),
    )(outp4, denp, h4flat, b4, xa3, Wv1, bv1, Wv2, bv2)
    return (px, vx)


# speedup vs baseline: 61.7741x; 1.4883x over previous
import functools

import jax
import jax.numpy as jnp
from jax import lax
from jax.experimental import pallas as pl
from jax.experimental.pallas import tpu as pltpu
from jax.experimental.pallas import tpu_sc as plsc

N = 10000
NPAD = 10112
E = 320000
NW = 32
RPT = 80
EPW = RPT * 128
EPAD = NW * EPW
ER = EPAD // 128
H = 16
GP = 8
NG = RPT // GP

_mesh = plsc.VectorSubcoreMesh(core_axis_name="c", subcore_axis_name="s")
_sc_params = pltpu.CompilerParams(needs_layout_passes=False,
                                  use_tc_tiling_on_sc=False)
f32 = jnp.float32
i32 = jnp.int32


def _wid_base():
    c = lax.axis_index("c")
    s = lax.axis_index("s")
    return c, s, (c * 16 + s) * RPT


@functools.partial(
    pl.kernel,
    out_type=(jax.ShapeDtypeStruct((2 * NPAD,), f32),
              jax.ShapeDtypeStruct((2 * NPAD,), f32),
              jax.ShapeDtypeStruct((2 * NPAD,), f32)),
    mesh=_mesh,
    compiler_params=_sc_params,
    scratch_types=[
        pltpu.VMEM((RPT, 128), i32),
        pltpu.VMEM((RPT, 128), f32),
        pltpu.VMEM((RPT, 128), f32),
        pltpu.VMEM((RPT, 128), f32),
        pltpu.VMEM_SHARED((NPAD,), f32),
        pltpu.VMEM_SHARED((NPAD,), f32),
        pltpu.VMEM_SHARED((NPAD,), f32),
        pltpu.SemaphoreType.DMA,
    ],
)
def _sc_pass0(dst_h, ew0_h, ew1_h, ones_h, zeros_h, outA_h, outB_h, outC_h,
              dstb, e0b, e1b, oneb, accA, accB, accC, ssem):
    c, s, base = _wid_base()

    @pl.when(s == 0)
    def _():
        pltpu.sync_copy(zeros_h, accA)
        pltpu.sync_copy(zeros_h, accB)
        pltpu.sync_copy(zeros_h, accC)

    pltpu.sync_copy(dst_h.at[pl.ds(base, RPT)], dstb)
    pltpu.sync_copy(ew0_h.at[pl.ds(base, RPT)], e0b)
    pltpu.sync_copy(ew1_h.at[pl.ds(base, RPT)], e1b)
    pltpu.sync_copy(ones_h, oneb)

    plsc.subcore_barrier()

    def sgroup(g, carry):
        hs = []
        for b in range(GP):
            j = g * GP + b
            hs.append(pltpu.async_copy(e0b.at[j], accA.at[dstb.at[j]], ssem,
                                       add=True))
            hs.append(pltpu.async_copy(e1b.at[j], accB.at[dstb.at[j]], ssem,
                                       add=True))
            hs.append(pltpu.async_copy(oneb.at[j], accC.at[dstb.at[j]], ssem,
                                       add=True))
        for hd in hs:
            hd.wait()
        return carry

    lax.fori_loop(0, NG, sgroup, 0)
    plsc.subcore_barrier()

    @pl.when(s == 0)
    def _():
        pltpu.sync_copy(accA, outA_h.at[pl.ds(c * NPAD, NPAD)])
        pltpu.sync_copy(accB, outB_h.at[pl.ds(c * NPAD, NPAD)])
        pltpu.sync_copy(accC, outC_h.at[pl.ds(c * NPAD, NPAD)])


@functools.partial(
    pl.kernel,
    out_type=(jax.ShapeDtypeStruct((2 * NPAD,), f32),
              jax.ShapeDtypeStruct((2, NPAD, H), f32)),
    mesh=_mesh,
    compiler_params=_sc_params,
    scratch_types=[
        pltpu.VMEM((RPT, 128), i32),
        pltpu.VMEM((RPT, 128), i32),
        pltpu.VMEM((RPT, 128), f32),
        pltpu.VMEM((RPT, 128), f32),
        pltpu.VMEM((RPT, 128), f32),
        pltpu.VMEM((NPAD,), f32),
        pltpu.VMEM((NPAD,), f32),
        pltpu.VMEM((NPAD,), f32),
        pltpu.VMEM((2, 16), f32),
        pltpu.VMEM((GP, 128, H), f32),
        pltpu.VMEM_SHARED((NPAD,), f32),
        pltpu.VMEM_SHARED((NPAD, H), f32),
        pltpu.SemaphoreType.DMA,
        pltpu.SemaphoreType.DMA,
    ],
)
def _sc_layer(src_h, dst_h, ew0_h, ew1_h, ssrc_h, sdst_h, msl_h, cb_h, h_h,
              zeros_h, zeros2_h,
              denp_h, outp_h,
              srcb, dstb, e0b, e1b, exb, ssrcT, sdstT, mslT, cbb, rows8,
              den_sh, out_sh, gsem, ssem):
    c, s, base = _wid_base()

    @pl.when(s == 0)
    def _():
        pltpu.sync_copy(zeros_h, den_sh)
        pltpu.sync_copy(zeros2_h, out_sh)

    pltpu.sync_copy(src_h.at[pl.ds(base, RPT)], srcb)
    pltpu.sync_copy(dst_h.at[pl.ds(base, RPT)], dstb)
    pltpu.sync_copy(ew0_h.at[pl.ds(base, RPT)], e0b)
    pltpu.sync_copy(ew1_h.at[pl.ds(base, RPT)], e1b)
    pltpu.sync_copy(ssrc_h, ssrcT)
    pltpu.sync_copy(sdst_h, sdstT)
    pltpu.sync_copy(msl_h, mslT)
    pltpu.sync_copy(cb_h, cbb)

    c0 = cbb[0, :]
    c1 = cbb[1, :]

    def vrow(j, carry):
        def vcol(k, carry2):
            sl = pl.ds(k * 16, 16)
            s16 = srcb[j, sl]
            d16 = dstb[j, sl]
            e0 = e0b[j, sl]
            e1 = e1b[j, sl]
            g1 = plsc.load_gather(ssrcT, [s16])
            g2 = plsc.load_gather(sdstT, [d16])
            g3 = plsc.load_gather(mslT, [d16])
            al = g1 + g2 + e0 * c0 + e1 * c1
            al = jnp.maximum(al, al * 0.2)
            t = jnp.minimum(al - g3, 60.0)
            exb[j, sl] = jnp.exp(t)
            return carry2
        lax.fori_loop(0, 8, vcol, 0)
        return carry

    lax.fori_loop(0, RPT, vrow, 0)
    plsc.subcore_barrier()

    zi = jnp.zeros((16,), i32)

    def group(g, carry):
        dh = []
        gh = []
        for b in range(GP):
            j = g * GP + b
            dh.append(pltpu.async_copy(exb.at[j], den_sh.at[dstb.at[j]], ssem,
                                       add=True))
            gh.append(pltpu.async_copy(h_h.at[srcb.at[j]], rows8.at[b], gsem))
        sh = []
        for b in range(GP):
            j = g * GP + b
            gh[b].wait()
            jsplat = zi + j

            def sbody(i, cidx):
                gsc = plsc.load_gather(exb, [jsplat, cidx])
                rows8[b, i, :] = rows8[b, i, :] * gsc
                return cidx + 1

            lax.fori_loop(0, 128, sbody, zi)
            sh.append(pltpu.async_copy(rows8.at[b], out_sh.at[dstb.at[j]],
                                       ssem, add=True))
        for hd in dh:
            hd.wait()
        for hd in sh:
            hd.wait()
        return carry

    lax.fori_loop(0, NG, group, 0)
    plsc.subcore_barrier()

    @pl.when(s == 0)
    def _():
        pltpu.sync_copy(den_sh, denp_h.at[pl.ds(c * NPAD, NPAD)])
        pltpu.sync_copy(out_sh, outp_h.at[c])


@functools.partial(
    pl.kernel,
    out_type=(jax.ShapeDtypeStruct((2 * NPAD,), f32),
              jax.ShapeDtypeStruct((2 * NPAD,), f32)),
    mesh=_mesh,
    compiler_params=_sc_params,
    scratch_types=[
        pltpu.VMEM((RPT, 128), i32),
        pltpu.VMEM((RPT, 128), i32),
        pltpu.VMEM((RPT, 128), f32),
        pltpu.VMEM((RPT, 128), f32),
        pltpu.VMEM((RPT, 128), f32),
        pltpu.VMEM((RPT, 128), f32),
        pltpu.VMEM((NPAD,), f32),
        pltpu.VMEM((NPAD,), f32),
        pltpu.VMEM((NPAD,), f32),
        pltpu.VMEM((NPAD,), f32),
        pltpu.VMEM((2, 16), f32),
        pltpu.VMEM_SHARED((NPAD,), f32),
        pltpu.VMEM_SHARED((NPAD,), f32),
        pltpu.SemaphoreType.DMA,
    ],
)
def _sc_layer4(src_h, dst_h, ew0_h, ew1_h, ssrc_h, sdst_h, msl_h, cb_h, h1_h,
               zeros_h,
               denp_h, outp_h,
               srcb, dstb, e0b, e1b, exb, vb, ssrcT, sdstT, mslT, h1T, cbb,
               den_sh, out_sh, ssem):
    c, s, base = _wid_base()

    @pl.when(s == 0)
    def _():
        pltpu.sync_copy(zeros_h, den_sh)
        pltpu.sync_copy(zeros_h, out_sh)

    pltpu.sync_copy(src_h.at[pl.ds(base, RPT)], srcb)
    pltpu.sync_copy(dst_h.at[pl.ds(base, RPT)], dstb)
    pltpu.sync_copy(ew0_h.at[pl.ds(base, RPT)], e0b)
    pltpu.sync_copy(ew1_h.at[pl.ds(base, RPT)], e1b)
    pltpu.sync_copy(ssrc_h, ssrcT)
    pltpu.sync_copy(sdst_h, sdstT)
    pltpu.sync_copy(msl_h, mslT)
    pltpu.sync_copy(h1_h, h1T)
    pltpu.sync_copy(cb_h, cbb)

    c0 = cbb[0, :]
    c1 = cbb[1, :]

    def vrow(j, carry):
        def vcol(k, carry2):
            sl = pl.ds(k * 16, 16)
            s16 = srcb[j, sl]
            d16 = dstb[j, sl]
            e0 = e0b[j, sl]
            e1 = e1b[j, sl]
            g1 = plsc.load_gather(ssrcT, [s16])
            g2 = plsc.load_gather(sdstT, [d16])
            g3 = plsc.load_gather(mslT, [d16])
            hg = plsc.load_gather(h1T, [s16])
            al = g1 + g2 + e0 * c0 + e1 * c1
            al = jnp.maximum(al, al * 0.2)
            t = jnp.minimum(al - g3, 60.0)
            ex16 = jnp.exp(t)
            exb[j, sl] = ex16
            vb[j, sl] = ex16 * hg
            return carry2
        lax.fori_loop(0, 8, vcol, 0)
        return carry

    lax.fori_loop(0, RPT, vrow, 0)
    plsc.subcore_barrier()

    def sgroup(g, carry):
        hs = []
        for b in range(GP):
            j = g * GP + b
            hs.append(pltpu.async_copy(exb.at[j], den_sh.at[dstb.at[j]], ssem,
                                       add=True))
            hs.append(pltpu.async_copy(vb.at[j], out_sh.at[dstb.at[j]], ssem,
                                       add=True))
        for hd in hs:
            hd.wait()
        return carry

    lax.fori_loop(0, NG, sgroup, 0)
    plsc.subcore_barrier()

    @pl.when(s == 0)
    def _():
        pltpu.sync_copy(den_sh, denp_h.at[pl.ds(c * NPAD, NPAD)])
        pltpu.sync_copy(out_sh, outp_h.at[pl.ds(c * NPAD, NPAD)])


def _attn_scalars(h, a_s, a_d, le, ae, loop0, loop1):
    ssrc = jnp.sum(h * a_s, axis=-1)
    sdst = jnp.sum(h * a_d, axis=-1)
    cvec = le @ ae
    t = ssrc + sdst + loop0 * cvec[0] + loop1 * cvec[1]
    msl = jnp.maximum(t, t * 0.2)
    cb = jnp.broadcast_to(cvec[:, None], (2, 16))
    return ssrc, sdst, msl, cb


def _tc_prep1(xp_ref, W1_ref, as1_ref, ad1_ref, le1_ref, ae1_ref,
              pA_ref, pB_ref, pC_ref,
              h_ref, ssrc_ref, sdst_ref, msl_ref, cb_ref, l0_ref, l1_ref):
    xp = xp_ref[...]
    h = jax.lax.dot(xp, W1_ref[...], preferred_element_type=f32)
    pA = pA_ref[...]
    pB = pB_ref[...]
    pC = pC_ref[...]
    cnt = jnp.maximum(pC[:NPAD] + pC[NPAD:], 1.0)
    loop0 = (pA[:NPAD] + pA[NPAD:]) / cnt
    loop1 = (pB[:NPAD] + pB[NPAD:]) / cnt
    ssrc, sdst, msl, cb = _attn_scalars(
        h, as1_ref[...], ad1_ref[...], le1_ref[...], ae1_ref[...], loop0, loop1)
    h_ref[...] = h
    ssrc_ref[...] = ssrc
    sdst_ref[...] = sdst
    msl_ref[...] = msl
    cb_ref[...] = cb
    l0_ref[...] = loop0
    l1_ref[...] = loop1


def _tc_fin(outp_ref, denp_ref, h_ref, b_ref, xa_ref):
    denp = denp_ref[...]
    den = denp[:NPAD] + denp[NPAD:] + 1.0
    xa = (outp_ref[0] + outp_ref[1] + h_ref[...]) / den[:, None]
    xa = jnp.maximum(xa + b_ref[...], 0.0)
    rmask = lax.broadcasted_iota(i32, (NPAD, H), 0) < N
    xa_ref[...] = jnp.where(rmask, xa, 0.0)


def _tc_prepn(xa_ref, xp_ref, W_ref, as_ref, ad_ref, le_ref, ae_ref,
              l0_ref, l1_ref,
              hn_ref, ssrc_ref, sdst_ref, msl_ref, cb_ref):
    hn = (jax.lax.dot(xa_ref[...], W_ref[0:H, :], preferred_element_type=f32)
          + jax.lax.dot(xp_ref[...], W_ref[H:, :], preferred_element_type=f32))
    ssrc, sdst, msl, cb = _attn_scalars(
        hn, as_ref[...], ad_ref[...], le_ref[...], ae_ref[...],
        l0_ref[...], l1_ref[...])
    hn_ref[...] = hn
    ssrc_ref[...] = ssrc
    sdst_ref[...] = sdst
    msl_ref[...] = msl
    cb_ref[...] = cb


def _tc_prep4(xa_ref, xp_ref, W_ref, as_ref, ad_ref, le_ref, ae_ref,
              l0_ref, l1_ref,
              hn_ref, ssrc_ref, sdst_ref, msl_ref, cb_ref):
    v1 = W_ref[0:H, 0]
    v2 = W_ref[H:, 0]
    hn = jnp.sum(xa_ref[...] * v1, axis=-1) + jnp.sum(xp_ref[...] * v2, axis=-1)
    ssrc = hn * as_ref[0]
    sdst = hn * ad_ref[0]
    cvec = le_ref[:, 0] * ae_ref[0]
    t = ssrc + sdst + l0_ref[...] * cvec[0] + l1_ref[...] * cvec[1]
    msl = jnp.maximum(t, t * 0.2)
    hn_ref[...] = hn
    ssrc_ref[...] = ssrc
    sdst_ref[...] = sdst
    msl_ref[...] = msl
    cb_ref[...] = jnp.broadcast_to(cvec[:, None], (2, 16))


def _tc_head(outp4_ref, denp4_ref, h4f_ref, b4_ref, xa3_ref,
             Wv1_ref, bv1_ref, Wv2_ref, bv2_ref,
             px_ref, vx_ref):
    denp4 = denp4_ref[...]
    outp4 = outp4_ref[...]
    den = denp4[:NPAD] + denp4[NPAD:] + 1.0
    p = (outp4[:NPAD] + outp4[NPAD:] + h4f_ref[...]) / den + b4_ref[0]
    p = jnp.maximum(p, 0.0)
    px_ref[...] = p[:N][None, :]
    v = jnp.sum(xa3_ref[...], axis=0) / float(N)
    vx = jnp.maximum(v @ Wv1_ref[...] + bv1_ref[...], 0.0)
    vx_ref[...] = (vx @ Wv2_ref[...] + bv2_ref[...])[None, :]


def _pc(fn, out_shape):
    return pl.pallas_call(fn, out_shape=out_shape)


def kernel(x, edge_index, edge_attr, W1, as1, ad1, ae1, le1, b1,
           W2, as2, ad2, ae2, le2, b2, W3, as3, ad3, ae3, le3, b3,
           W4, as4, ad4, ae4, le4, b4, Wv1, bv1, Wv2, bv2):
    src = edge_index[0]
    dst = edge_index[1]
    pad_e = EPAD - E
    padi = jnp.full((pad_e,), N, i32)
    src2 = jnp.concatenate([src, padi]).reshape(ER, 128)
    dst2 = jnp.concatenate([dst, padi]).reshape(ER, 128)
    padf = jnp.zeros((pad_e,), f32)
    ew0 = jnp.concatenate([edge_attr[:, 0], padf]).reshape(ER, 128)
    ew1 = jnp.concatenate([edge_attr[:, 1], padf]).reshape(ER, 128)
    xp = jnp.pad(x, ((0, NPAD - N), (0, 0)))
    zN = jnp.zeros((NPAD,), f32)
    zNH = jnp.zeros((NPAD, H), f32)
    onesR = jnp.ones((RPT, 128), f32)

    pA, pB, pC = _sc_pass0(dst2, ew0, ew1, onesR, zN)

    sds = jax.ShapeDtypeStruct
    h1, ssrc, sdst, msl, cb, loop0, loop1 = _pc(
        _tc_prep1,
        (sds((NPAD, H), f32), sds((NPAD,), f32), sds((NPAD,), f32),
         sds((NPAD,), f32), sds((2, 16), f32), sds((NPAD,), f32),
         sds((NPAD,), f32)),
    )(xp, W1, as1, ad1, le1, ae1, pA, pB, pC)

    layers = [
        (b1, W2, as2, ad2, le2, ae2),
        (b2, W3, as3, ad3, le3, ae3),
        (b3, W4, as4, ad4, le4, ae4),
    ]

    h_cur = h1
    xa3 = None
    h4flat = None
    for li, (b_l, Wn, asn, adn, len_, aen) in enumerate(layers):
        denp, outp = _sc_layer(src2, dst2, ew0, ew1, ssrc, sdst, msl, cb,
                               h_cur, zN, zNH)
        xa = _pc(_tc_fin, sds((NPAD, H), f32))(outp, denp, h_cur, b_l)
        if li < 2:
            h_cur, ssrc, sdst, msl, cb = _pc(
                _tc_prepn,
                (sds((NPAD, H), f32), sds((NPAD,), f32), sds((NPAD,), f32),
                 sds((NPAD,), f32), sds((2, 16), f32)),
            )(xa, xp, Wn, asn, adn, len_, aen, loop0, loop1)
        else:
            h4flat, ssrc, sdst, msl, cb = _pc(
                _tc_prep4,
                (sds((NPAD,), f32), sds((NPAD,), f32), sds((NPAD,), f32),
                 sds((NPAD,), f32), sds((2, 16), f32)),
            )(xa, xp, Wn, asn, adn, len_, aen, loop0, loop1)
            xa3 = xa

    denp4, outp4 = _sc_layer4(src2, dst2, ew0, ew1, ssrc, sdst, msl, cb,
                              h4flat, zN)

    px, vx = _pc(
        _tc_head,
        (sds((1, N), f32), sds((1, 1), f32)),
    )(outp4, denp4, h4flat, b4, xa3, Wv1, bv1, Wv2, bv2)
    return (px, vx)


# speedup vs baseline: 76.6953x; 1.2415x over previous
import functools

import jax
import jax.numpy as jnp
from jax import lax
from jax.experimental import pallas as pl
from jax.experimental.pallas import tpu as pltpu
from jax.experimental.pallas import tpu_sc as plsc

N = 10000
NPAD = 10112
E = 320000
NW = 32
RPT = 80
EPW = RPT * 128
EPAD = NW * EPW
ER = EPAD // 128
H = 16
GP = 8
NG = RPT // GP

_mesh = plsc.VectorSubcoreMesh(core_axis_name="c", subcore_axis_name="s")
_sc_params = pltpu.CompilerParams(needs_layout_passes=False,
                                  use_tc_tiling_on_sc=False)
f32 = jnp.float32
i32 = jnp.int32


def _wid_base():
    c = lax.axis_index("c")
    s = lax.axis_index("s")
    return c, s, (c * 16 + s) * RPT


@functools.partial(
    pl.kernel,
    out_type=(jax.ShapeDtypeStruct((2 * NPAD,), f32),
              jax.ShapeDtypeStruct((2 * NPAD,), f32),
              jax.ShapeDtypeStruct((2 * NPAD,), f32)),
    mesh=_mesh,
    compiler_params=_sc_params,
    scratch_types=[
        pltpu.VMEM((RPT, 128), i32),
        pltpu.VMEM((RPT, 128), f32),
        pltpu.VMEM((RPT, 128), f32),
        pltpu.VMEM((RPT, 128), f32),
        pltpu.VMEM_SHARED((NPAD,), f32),
        pltpu.VMEM_SHARED((NPAD,), f32),
        pltpu.VMEM_SHARED((NPAD,), f32),
        pltpu.SemaphoreType.DMA,
    ],
)
def _sc_pass0(dst_h, ew0_h, ew1_h, ones_h, zeros_h, outA_h, outB_h, outC_h,
              dstb, e0b, e1b, oneb, accA, accB, accC, ssem):
    c, s, base = _wid_base()

    @pl.when(s == 0)
    def _():
        pltpu.sync_copy(zeros_h, accA)
        pltpu.sync_copy(zeros_h, accB)
        pltpu.sync_copy(zeros_h, accC)

    pltpu.sync_copy(dst_h.at[pl.ds(base, RPT)], dstb)
    pltpu.sync_copy(ew0_h.at[pl.ds(base, RPT)], e0b)
    pltpu.sync_copy(ew1_h.at[pl.ds(base, RPT)], e1b)
    pltpu.sync_copy(ones_h, oneb)

    plsc.subcore_barrier()

    def sgroup(g, carry):
        hs = []
        for b in range(GP):
            j = g * GP + b
            hs.append(pltpu.async_copy(e0b.at[j], accA.at[dstb.at[j]], ssem,
                                       add=True))
            hs.append(pltpu.async_copy(e1b.at[j], accB.at[dstb.at[j]], ssem,
                                       add=True))
            hs.append(pltpu.async_copy(oneb.at[j], accC.at[dstb.at[j]], ssem,
                                       add=True))
        for hd in hs:
            hd.wait()
        return carry

    lax.fori_loop(0, NG, sgroup, 0)
    plsc.subcore_barrier()

    @pl.when(s == 0)
    def _():
        pltpu.sync_copy(accA, outA_h.at[pl.ds(c * NPAD, NPAD)])
        pltpu.sync_copy(accB, outB_h.at[pl.ds(c * NPAD, NPAD)])
        pltpu.sync_copy(accC, outC_h.at[pl.ds(c * NPAD, NPAD)])


@functools.partial(
    pl.kernel,
    out_type=(jax.ShapeDtypeStruct((2 * NPAD,), f32),
              jax.ShapeDtypeStruct((2, NPAD, H), f32)),
    mesh=_mesh,
    compiler_params=_sc_params,
    scratch_types=[
        pltpu.VMEM((RPT, 128), i32),
        pltpu.VMEM((RPT, 128), i32),
        pltpu.VMEM((RPT, 128), f32),
        pltpu.VMEM((RPT, 128), f32),
        pltpu.VMEM((RPT, 128), f32),
        pltpu.VMEM((NPAD,), f32),
        pltpu.VMEM((NPAD,), f32),
        pltpu.VMEM((NPAD,), f32),
        pltpu.VMEM((2, 16), f32),
        pltpu.VMEM((GP, 128, H), f32),
        pltpu.VMEM_SHARED((NPAD,), f32),
        pltpu.VMEM_SHARED((NPAD, H), f32),
        pltpu.SemaphoreType.DMA,
        pltpu.SemaphoreType.DMA,
    ],
)
def _sc_layer(src_h, dst_h, ew0_h, ew1_h, ssrc_h, sdst_h, msl_h, cb_h, h_h,
              zeros_h, zeros2_h,
              denp_h, outp_h,
              srcb, dstb, e0b, e1b, exb, ssrcT, sdstT, mslT, cbb, rows8,
              den_sh, out_sh, gsem, ssem):
    c, s, base = _wid_base()

    @pl.when(s == 0)
    def _():
        pltpu.sync_copy(zeros_h, den_sh)
        pltpu.sync_copy(zeros2_h, out_sh)

    pltpu.sync_copy(src_h.at[pl.ds(base, RPT)], srcb)
    pltpu.sync_copy(dst_h.at[pl.ds(base, RPT)], dstb)
    pltpu.sync_copy(ew0_h.at[pl.ds(base, RPT)], e0b)
    pltpu.sync_copy(ew1_h.at[pl.ds(base, RPT)], e1b)
    pltpu.sync_copy(ssrc_h, ssrcT)
    pltpu.sync_copy(sdst_h, sdstT)
    pltpu.sync_copy(msl_h, mslT)
    pltpu.sync_copy(cb_h, cbb)

    c0 = cbb[0, :]
    c1 = cbb[1, :]

    @plsc.parallel_loop(0, RPT, 1, unroll=2)
    def _(j):
        for k in range(8):
            sl = pl.ds(k * 16, 16)
            s16 = srcb[j, sl]
            d16 = dstb[j, sl]
            e0 = e0b[j, sl]
            e1 = e1b[j, sl]
            g1 = plsc.load_gather(ssrcT, [s16])
            g2 = plsc.load_gather(sdstT, [d16])
            g3 = plsc.load_gather(mslT, [d16])
            al = g1 + g2 + e0 * c0 + e1 * c1
            al = jnp.maximum(al, al * 0.2)
            t = jnp.minimum(al - g3, 60.0)
            exb[j, sl] = jnp.exp(t)

    plsc.subcore_barrier()

    zi = jnp.zeros((16,), i32)

    def group(g, carry):
        dh = []
        gh = []
        for b in range(GP):
            j = g * GP + b
            dh.append(pltpu.async_copy(exb.at[j], den_sh.at[dstb.at[j]], ssem,
                                       add=True))
            gh.append(pltpu.async_copy(h_h.at[srcb.at[j]], rows8.at[b], gsem))
        sh = []
        for b in range(GP):
            j = g * GP + b
            gh[b].wait()
            jsplat = zi + j

            @plsc.parallel_loop(0, 128, 1, unroll=8)
            def _(i):
                gsc = plsc.load_gather(exb, [jsplat, zi + i])
                rows8[b, i, :] = rows8[b, i, :] * gsc
            sh.append(pltpu.async_copy(rows8.at[b], out_sh.at[dstb.at[j]],
                                       ssem, add=True))
        for hd in dh:
            hd.wait()
        for hd in sh:
            hd.wait()
        return carry

    lax.fori_loop(0, NG, group, 0)
    plsc.subcore_barrier()

    @pl.when(s == 0)
    def _():
        pltpu.sync_copy(den_sh, denp_h.at[pl.ds(c * NPAD, NPAD)])
        pltpu.sync_copy(out_sh, outp_h.at[c])


@functools.partial(
    pl.kernel,
    out_type=(jax.ShapeDtypeStruct((2 * NPAD,), f32),
              jax.ShapeDtypeStruct((2 * NPAD,), f32)),
    mesh=_mesh,
    compiler_params=_sc_params,
    scratch_types=[
        pltpu.VMEM((RPT, 128), i32),
        pltpu.VMEM((RPT, 128), i32),
        pltpu.VMEM((RPT, 128), f32),
        pltpu.VMEM((RPT, 128), f32),
        pltpu.VMEM((RPT, 128), f32),
        pltpu.VMEM((RPT, 128), f32),
        pltpu.VMEM((NPAD,), f32),
        pltpu.VMEM((NPAD,), f32),
        pltpu.VMEM((NPAD,), f32),
        pltpu.VMEM((NPAD,), f32),
        pltpu.VMEM((2, 16), f32),
        pltpu.VMEM_SHARED((NPAD,), f32),
        pltpu.VMEM_SHARED((NPAD,), f32),
        pltpu.SemaphoreType.DMA,
    ],
)
def _sc_layer4(src_h, dst_h, ew0_h, ew1_h, ssrc_h, sdst_h, msl_h, cb_h, h1_h,
               zeros_h,
               denp_h, outp_h,
               srcb, dstb, e0b, e1b, exb, vb, ssrcT, sdstT, mslT, h1T, cbb,
               den_sh, out_sh, ssem):
    c, s, base = _wid_base()

    @pl.when(s == 0)
    def _():
        pltpu.sync_copy(zeros_h, den_sh)
        pltpu.sync_copy(zeros_h, out_sh)

    pltpu.sync_copy(src_h.at[pl.ds(base, RPT)], srcb)
    pltpu.sync_copy(dst_h.at[pl.ds(base, RPT)], dstb)
    pltpu.sync_copy(ew0_h.at[pl.ds(base, RPT)], e0b)
    pltpu.sync_copy(ew1_h.at[pl.ds(base, RPT)], e1b)
    pltpu.sync_copy(ssrc_h, ssrcT)
    pltpu.sync_copy(sdst_h, sdstT)
    pltpu.sync_copy(msl_h, mslT)
    pltpu.sync_copy(h1_h, h1T)
    pltpu.sync_copy(cb_h, cbb)

    c0 = cbb[0, :]
    c1 = cbb[1, :]

    @plsc.parallel_loop(0, RPT, 1, unroll=2)
    def _(j):
        for k in range(8):
            sl = pl.ds(k * 16, 16)
            s16 = srcb[j, sl]
            d16 = dstb[j, sl]
            e0 = e0b[j, sl]
            e1 = e1b[j, sl]
            g1 = plsc.load_gather(ssrcT, [s16])
            g2 = plsc.load_gather(sdstT, [d16])
            g3 = plsc.load_gather(mslT, [d16])
            hg = plsc.load_gather(h1T, [s16])
            al = g1 + g2 + e0 * c0 + e1 * c1
            al = jnp.maximum(al, al * 0.2)
            t = jnp.minimum(al - g3, 60.0)
            ex16 = jnp.exp(t)
            exb[j, sl] = ex16
            vb[j, sl] = ex16 * hg

    plsc.subcore_barrier()

    def sgroup(g, carry):
        hs = []
        for b in range(GP):
            j = g * GP + b
            hs.append(pltpu.async_copy(exb.at[j], den_sh.at[dstb.at[j]], ssem,
                                       add=True))
            hs.append(pltpu.async_copy(vb.at[j], out_sh.at[dstb.at[j]], ssem,
                                       add=True))
        for hd in hs:
            hd.wait()
        return carry

    lax.fori_loop(0, NG, sgroup, 0)
    plsc.subcore_barrier()

    @pl.when(s == 0)
    def _():
        pltpu.sync_copy(den_sh, denp_h.at[pl.ds(c * NPAD, NPAD)])
        pltpu.sync_copy(out_sh, outp_h.at[pl.ds(c * NPAD, NPAD)])


def _attn_scalars(h, a_s, a_d, le, ae, loop0, loop1):
    ssrc = jnp.sum(h * a_s, axis=-1)
    sdst = jnp.sum(h * a_d, axis=-1)
    cvec = le @ ae
    t = ssrc + sdst + loop0 * cvec[0] + loop1 * cvec[1]
    msl = jnp.maximum(t, t * 0.2)
    cb = jnp.broadcast_to(cvec[:, None], (2, 16))
    return ssrc, sdst, msl, cb


def _tc_prep1(xp_ref, W1_ref, as1_ref, ad1_ref, le1_ref, ae1_ref,
              pA_ref, pB_ref, pC_ref,
              h_ref, ssrc_ref, sdst_ref, msl_ref, cb_ref, l0_ref, l1_ref):
    xp = xp_ref[...]
    h = jax.lax.dot(xp, W1_ref[...], preferred_element_type=f32)
    pA = pA_ref[...]
    pB = pB_ref[...]
    pC = pC_ref[...]
    cnt = jnp.maximum(pC[:NPAD] + pC[NPAD:], 1.0)
    loop0 = (pA[:NPAD] + pA[NPAD:]) / cnt
    loop1 = (pB[:NPAD] + pB[NPAD:]) / cnt
    ssrc, sdst, msl, cb = _attn_scalars(
        h, as1_ref[...], ad1_ref[...], le1_ref[...], ae1_ref[...], loop0, loop1)
    h_ref[...] = h
    ssrc_ref[...] = ssrc
    sdst_ref[...] = sdst
    msl_ref[...] = msl
    cb_ref[...] = cb
    l0_ref[...] = loop0
    l1_ref[...] = loop1


def _tc_fin(outp_ref, denp_ref, h_ref, b_ref, xa_ref):
    denp = denp_ref[...]
    den = denp[:NPAD] + denp[NPAD:] + 1.0
    xa = (outp_ref[0] + outp_ref[1] + h_ref[...]) / den[:, None]
    xa = jnp.maximum(xa + b_ref[...], 0.0)
    rmask = lax.broadcasted_iota(i32, (NPAD, H), 0) < N
    xa_ref[...] = jnp.where(rmask, xa, 0.0)


def _tc_prepn(xa_ref, xp_ref, W_ref, as_ref, ad_ref, le_ref, ae_ref,
              l0_ref, l1_ref,
              hn_ref, ssrc_ref, sdst_ref, msl_ref, cb_ref):
    hn = (jax.lax.dot(xa_ref[...], W_ref[0:H, :], preferred_element_type=f32)
          + jax.lax.dot(xp_ref[...], W_ref[H:, :], preferred_element_type=f32))
    ssrc, sdst, msl, cb = _attn_scalars(
        hn, as_ref[...], ad_ref[...], le_ref[...], ae_ref[...],
        l0_ref[...], l1_ref[...])
    hn_ref[...] = hn
    ssrc_ref[...] = ssrc
    sdst_ref[...] = sdst
    msl_ref[...] = msl
    cb_ref[...] = cb


def _tc_prep4(xa_ref, xp_ref, W_ref, as_ref, ad_ref, le_ref, ae_ref,
              l0_ref, l1_ref,
              hn_ref, ssrc_ref, sdst_ref, msl_ref, cb_ref):
    v1 = W_ref[0:H, 0]
    v2 = W_ref[H:, 0]
    hn = jnp.sum(xa_ref[...] * v1, axis=-1) + jnp.sum(xp_ref[...] * v2, axis=-1)
    ssrc = hn * as_ref[0]
    sdst = hn * ad_ref[0]
    cvec = le_ref[:, 0] * ae_ref[0]
    t = ssrc + sdst + l0_ref[...] * cvec[0] + l1_ref[...] * cvec[1]
    msl = jnp.maximum(t, t * 0.2)
    hn_ref[...] = hn
    ssrc_ref[...] = ssrc
    sdst_ref[...] = sdst
    msl_ref[...] = msl
    cb_ref[...] = jnp.broadcast_to(cvec[:, None], (2, 16))


def _tc_head(outp4_ref, denp4_ref, h4f_ref, b4_ref, xa3_ref,
             Wv1_ref, bv1_ref, Wv2_ref, bv2_ref,
             px_ref, vx_ref):
    denp4 = denp4_ref[...]
    outp4 = outp4_ref[...]
    den = denp4[:NPAD] + denp4[NPAD:] + 1.0
    p = (outp4[:NPAD] + outp4[NPAD:] + h4f_ref[...]) / den + b4_ref[0]
    p = jnp.maximum(p, 0.0)
    px_ref[...] = p[:N][None, :]
    v = jnp.sum(xa3_ref[...], axis=0) / float(N)
    vx = jnp.maximum(v @ Wv1_ref[...] + bv1_ref[...], 0.0)
    vx_ref[...] = (vx @ Wv2_ref[...] + bv2_ref[...])[None, :]


def _pc(fn, out_shape):
    return pl.pallas_call(fn, out_shape=out_shape)


def kernel(x, edge_index, edge_attr, W1, as1, ad1, ae1, le1, b1,
           W2, as2, ad2, ae2, le2, b2, W3, as3, ad3, ae3, le3, b3,
           W4, as4, ad4, ae4, le4, b4, Wv1, bv1, Wv2, bv2):
    src = edge_index[0]
    dst = edge_index[1]
    pad_e = EPAD - E
    padi = jnp.full((pad_e,), N, i32)
    src2 = jnp.concatenate([src, padi]).reshape(ER, 128)
    dst2 = jnp.concatenate([dst, padi]).reshape(ER, 128)
    padf = jnp.zeros((pad_e,), f32)
    ew0 = jnp.concatenate([edge_attr[:, 0], padf]).reshape(ER, 128)
    ew1 = jnp.concatenate([edge_attr[:, 1], padf]).reshape(ER, 128)
    xp = jnp.pad(x, ((0, NPAD - N), (0, 0)))
    zN = jnp.zeros((NPAD,), f32)
    zNH = jnp.zeros((NPAD, H), f32)
    onesR = jnp.ones((RPT, 128), f32)

    pA, pB, pC = _sc_pass0(dst2, ew0, ew1, onesR, zN)

    sds = jax.ShapeDtypeStruct
    h1, ssrc, sdst, msl, cb, loop0, loop1 = _pc(
        _tc_prep1,
        (sds((NPAD, H), f32), sds((NPAD,), f32), sds((NPAD,), f32),
         sds((NPAD,), f32), sds((2, 16), f32), sds((NPAD,), f32),
         sds((NPAD,), f32)),
    )(xp, W1, as1, ad1, le1, ae1, pA, pB, pC)

    layers = [
        (b1, W2, as2, ad2, le2, ae2),
        (b2, W3, as3, ad3, le3, ae3),
        (b3, W4, as4, ad4, le4, ae4),
    ]

    h_cur = h1
    xa3 = None
    h4flat = None
    for li, (b_l, Wn, asn, adn, len_, aen) in enumerate(layers):
        denp, outp = _sc_layer(src2, dst2, ew0, ew1, ssrc, sdst, msl, cb,
                               h_cur, zN, zNH)
        xa = _pc(_tc_fin, sds((NPAD, H), f32))(outp, denp, h_cur, b_l)
        if li < 2:
            h_cur, ssrc, sdst, msl, cb = _pc(
                _tc_prepn,
                (sds((NPAD, H), f32), sds((NPAD,), f32), sds((NPAD,), f32),
                 sds((NPAD,), f32), sds((2, 16), f32)),
            )(xa, xp, Wn, asn, adn, len_, aen, loop0, loop1)
        else:
            h4flat, ssrc, sdst, msl, cb = _pc(
                _tc_prep4,
                (sds((NPAD,), f32), sds((NPAD,), f32), sds((NPAD,), f32),
                 sds((NPAD,), f32), sds((2, 16), f32)),
            )(xa, xp, Wn, asn, adn, len_, aen, loop0, loop1)
            xa3 = xa

    denp4, outp4 = _sc_layer4(src2, dst2, ew0, ew1, ssrc, sdst, msl, cb,
                              h4flat, zN)

    px, vx = _pc(
        _tc_head,
        (sds((1, N), f32), sds((1, 1), f32)),
    )(outp4, denp4, h4flat, b4, xa3, Wv1, bv1, Wv2, bv2)
    return (px, vx)


# speedup vs baseline: 77.1931x; 1.0065x over previous
import functools

import jax
import jax.numpy as jnp
from jax import lax
from jax.experimental import pallas as pl
from jax.experimental.pallas import tpu as pltpu
from jax.experimental.pallas import tpu_sc as plsc

N = 10000
NPAD = 10112
E = 320000
NW = 32
RPT = 80
EPW = RPT * 128
EPAD = NW * EPW
ER = EPAD // 128
H = 16
GP = 8
NG = RPT // GP

_mesh = plsc.VectorSubcoreMesh(core_axis_name="c", subcore_axis_name="s")
_sc_params = pltpu.CompilerParams(needs_layout_passes=False,
                                  use_tc_tiling_on_sc=False)
f32 = jnp.float32
i32 = jnp.int32


def _wid_base():
    c = lax.axis_index("c")
    s = lax.axis_index("s")
    return c, s, (c * 16 + s) * RPT


@functools.partial(
    pl.kernel,
    out_type=(jax.ShapeDtypeStruct((2 * NPAD,), f32),
              jax.ShapeDtypeStruct((2 * NPAD,), f32),
              jax.ShapeDtypeStruct((2 * NPAD,), f32)),
    mesh=_mesh,
    compiler_params=_sc_params,
    scratch_types=[
        pltpu.VMEM((RPT, 128), i32),
        pltpu.VMEM((RPT, 128), f32),
        pltpu.VMEM((RPT, 128), f32),
        pltpu.VMEM((RPT, 128), f32),
        pltpu.VMEM_SHARED((NPAD,), f32),
        pltpu.VMEM_SHARED((NPAD,), f32),
        pltpu.VMEM_SHARED((NPAD,), f32),
        pltpu.SemaphoreType.DMA,
    ],
)
def _sc_pass0(dst_h, ew0_h, ew1_h, ones_h, zeros_h, outA_h, outB_h, outC_h,
              dstb, e0b, e1b, oneb, accA, accB, accC, ssem):
    c, s, base = _wid_base()

    @pl.when(s == 0)
    def _():
        pltpu.sync_copy(zeros_h, accA)
        pltpu.sync_copy(zeros_h, accB)
        pltpu.sync_copy(zeros_h, accC)

    pltpu.sync_copy(dst_h.at[pl.ds(base, RPT)], dstb)
    pltpu.sync_copy(ew0_h.at[pl.ds(base, RPT)], e0b)
    pltpu.sync_copy(ew1_h.at[pl.ds(base, RPT)], e1b)
    pltpu.sync_copy(ones_h, oneb)

    plsc.subcore_barrier()

    hs = {}
    for g in range(NG):
        if g >= 2:
            for hd in hs[g - 2]:
                hd.wait()
        hs[g] = []
        for b in range(GP):
            j = g * GP + b
            hs[g].append(pltpu.async_copy(e0b.at[j], accA.at[dstb.at[j]],
                                          ssem, add=True))
            hs[g].append(pltpu.async_copy(e1b.at[j], accB.at[dstb.at[j]],
                                          ssem, add=True))
            hs[g].append(pltpu.async_copy(oneb.at[j], accC.at[dstb.at[j]],
                                          ssem, add=True))
    for g in (NG - 2, NG - 1):
        for hd in hs[g]:
            hd.wait()
    plsc.subcore_barrier()

    @pl.when(s == 0)
    def _():
        pltpu.sync_copy(accA, outA_h.at[pl.ds(c * NPAD, NPAD)])
        pltpu.sync_copy(accB, outB_h.at[pl.ds(c * NPAD, NPAD)])
        pltpu.sync_copy(accC, outC_h.at[pl.ds(c * NPAD, NPAD)])


@functools.partial(
    pl.kernel,
    out_type=(jax.ShapeDtypeStruct((2 * NPAD,), f32),
              jax.ShapeDtypeStruct((2, NPAD, H), f32)),
    mesh=_mesh,
    compiler_params=_sc_params,
    scratch_types=[
        pltpu.VMEM((RPT, 128), i32),
        pltpu.VMEM((RPT, 128), i32),
        pltpu.VMEM((RPT, 128), f32),
        pltpu.VMEM((RPT, 128), f32),
        pltpu.VMEM((RPT, 128), f32),
        pltpu.VMEM((NPAD,), f32),
        pltpu.VMEM((NPAD,), f32),
        pltpu.VMEM((NPAD,), f32),
        pltpu.VMEM((2, 16), f32),
        pltpu.VMEM((2, GP, 128, H), f32),
        pltpu.VMEM_SHARED((NPAD,), f32),
        pltpu.VMEM_SHARED((NPAD, H), f32),
        pltpu.SemaphoreType.DMA,
        pltpu.SemaphoreType.DMA,
    ],
)
def _sc_layer(src_h, dst_h, ew0_h, ew1_h, ssrc_h, sdst_h, msl_h, cb_h, h_h,
              zeros_h, zeros2_h,
              denp_h, outp_h,
              srcb, dstb, e0b, e1b, exb, ssrcT, sdstT, mslT, cbb, rows8,
              den_sh, out_sh, gsem, ssem):
    c, s, base = _wid_base()

    @pl.when(s == 0)
    def _():
        pltpu.sync_copy(zeros_h, den_sh)
        pltpu.sync_copy(zeros2_h, out_sh)

    pltpu.sync_copy(src_h.at[pl.ds(base, RPT)], srcb)
    pltpu.sync_copy(dst_h.at[pl.ds(base, RPT)], dstb)
    pltpu.sync_copy(ew0_h.at[pl.ds(base, RPT)], e0b)
    pltpu.sync_copy(ew1_h.at[pl.ds(base, RPT)], e1b)
    pltpu.sync_copy(ssrc_h, ssrcT)
    pltpu.sync_copy(sdst_h, sdstT)
    pltpu.sync_copy(msl_h, mslT)
    pltpu.sync_copy(cb_h, cbb)

    c0 = cbb[0, :]
    c1 = cbb[1, :]

    @plsc.parallel_loop(0, RPT, 1, unroll=2)
    def _(j):
        for k in range(8):
            sl = pl.ds(k * 16, 16)
            s16 = srcb[j, sl]
            d16 = dstb[j, sl]
            e0 = e0b[j, sl]
            e1 = e1b[j, sl]
            g1 = plsc.load_gather(ssrcT, [s16])
            g2 = plsc.load_gather(sdstT, [d16])
            g3 = plsc.load_gather(mslT, [d16])
            al = g1 + g2 + e0 * c0 + e1 * c1
            al = jnp.maximum(al, al * 0.2)
            t = jnp.minimum(al - g3, 60.0)
            exb[j, sl] = jnp.exp(t)

    plsc.subcore_barrier()

    zi = jnp.zeros((16,), i32)
    gh = {}
    sh = {}
    dh = {}

    def scale_scatter(g):
        bb = g % 2
        sh[g] = []
        for b in range(GP):
            j = g * GP + b
            gh[g][b].wait()
            jsplat = zi + j

            @plsc.parallel_loop(0, 128, 1, unroll=4)
            def _(i):
                gsc = plsc.load_gather(exb, [jsplat, zi + i])
                rows8[bb, b, i, :] = rows8[bb, b, i, :] * gsc

            sh[g].append(pltpu.async_copy(rows8.at[bb, b],
                                          out_sh.at[dstb.at[j]], ssem,
                                          add=True))

    for g in range(NG):
        if g >= 2:
            for hd in sh[g - 2]:
                hd.wait()
            for hd in dh[g - 2]:
                hd.wait()
        bb = g % 2
        dh[g] = []
        gh[g] = []
        for b in range(GP):
            j = g * GP + b
            dh[g].append(pltpu.async_copy(exb.at[j], den_sh.at[dstb.at[j]],
                                          ssem, add=True))
            gh[g].append(pltpu.async_copy(h_h.at[srcb.at[j]],
                                          rows8.at[bb, b], gsem))
        if g >= 1:
            scale_scatter(g - 1)
    scale_scatter(NG - 1)
    for g in (NG - 2, NG - 1):
        for hd in sh[g]:
            hd.wait()
        for hd in dh[g]:
            hd.wait()
    plsc.subcore_barrier()

    @pl.when(s == 0)
    def _():
        pltpu.sync_copy(den_sh, denp_h.at[pl.ds(c * NPAD, NPAD)])
        pltpu.sync_copy(out_sh, outp_h.at[c])


@functools.partial(
    pl.kernel,
    out_type=(jax.ShapeDtypeStruct((2 * NPAD,), f32),
              jax.ShapeDtypeStruct((2 * NPAD,), f32)),
    mesh=_mesh,
    compiler_params=_sc_params,
    scratch_types=[
        pltpu.VMEM((RPT, 128), i32),
        pltpu.VMEM((RPT, 128), i32),
        pltpu.VMEM((RPT, 128), f32),
        pltpu.VMEM((RPT, 128), f32),
        pltpu.VMEM((RPT, 128), f32),
        pltpu.VMEM((RPT, 128), f32),
        pltpu.VMEM((NPAD,), f32),
        pltpu.VMEM((NPAD,), f32),
        pltpu.VMEM((NPAD,), f32),
        pltpu.VMEM((NPAD,), f32),
        pltpu.VMEM((2, 16), f32),
        pltpu.VMEM_SHARED((NPAD,), f32),
        pltpu.VMEM_SHARED((NPAD,), f32),
        pltpu.SemaphoreType.DMA,
    ],
)
def _sc_layer4(src_h, dst_h, ew0_h, ew1_h, ssrc_h, sdst_h, msl_h, cb_h, h1_h,
               zeros_h,
               denp_h, outp_h,
               srcb, dstb, e0b, e1b, exb, vb, ssrcT, sdstT, mslT, h1T, cbb,
               den_sh, out_sh, ssem):
    c, s, base = _wid_base()

    @pl.when(s == 0)
    def _():
        pltpu.sync_copy(zeros_h, den_sh)
        pltpu.sync_copy(zeros_h, out_sh)

    pltpu.sync_copy(src_h.at[pl.ds(base, RPT)], srcb)
    pltpu.sync_copy(dst_h.at[pl.ds(base, RPT)], dstb)
    pltpu.sync_copy(ew0_h.at[pl.ds(base, RPT)], e0b)
    pltpu.sync_copy(ew1_h.at[pl.ds(base, RPT)], e1b)
    pltpu.sync_copy(ssrc_h, ssrcT)
    pltpu.sync_copy(sdst_h, sdstT)
    pltpu.sync_copy(msl_h, mslT)
    pltpu.sync_copy(h1_h, h1T)
    pltpu.sync_copy(cb_h, cbb)

    c0 = cbb[0, :]
    c1 = cbb[1, :]

    @plsc.parallel_loop(0, RPT, 1, unroll=2)
    def _(j):
        for k in range(8):
            sl = pl.ds(k * 16, 16)
            s16 = srcb[j, sl]
            d16 = dstb[j, sl]
            e0 = e0b[j, sl]
            e1 = e1b[j, sl]
            g1 = plsc.load_gather(ssrcT, [s16])
            g2 = plsc.load_gather(sdstT, [d16])
            g3 = plsc.load_gather(mslT, [d16])
            hg = plsc.load_gather(h1T, [s16])
            al = g1 + g2 + e0 * c0 + e1 * c1
            al = jnp.maximum(al, al * 0.2)
            t = jnp.minimum(al - g3, 60.0)
            ex16 = jnp.exp(t)
            exb[j, sl] = ex16
            vb[j, sl] = ex16 * hg

    plsc.subcore_barrier()

    hs = {}
    for g in range(NG):
        if g >= 2:
            for hd in hs[g - 2]:
                hd.wait()
        hs[g] = []
        for b in range(GP):
            j = g * GP + b
            hs[g].append(pltpu.async_copy(exb.at[j], den_sh.at[dstb.at[j]],
                                          ssem, add=True))
            hs[g].append(pltpu.async_copy(vb.at[j], out_sh.at[dstb.at[j]],
                                          ssem, add=True))
    for g in (NG - 2, NG - 1):
        for hd in hs[g]:
            hd.wait()
    plsc.subcore_barrier()

    @pl.when(s == 0)
    def _():
        pltpu.sync_copy(den_sh, denp_h.at[pl.ds(c * NPAD, NPAD)])
        pltpu.sync_copy(out_sh, outp_h.at[pl.ds(c * NPAD, NPAD)])


def _attn_scalars(h, a_s, a_d, le, ae, loop0, loop1):
    ssrc = jnp.sum(h * a_s, axis=-1)
    sdst = jnp.sum(h * a_d, axis=-1)
    cvec = le @ ae
    t = ssrc + sdst + loop0 * cvec[0] + loop1 * cvec[1]
    msl = jnp.maximum(t, t * 0.2)
    cb = jnp.broadcast_to(cvec[:, None], (2, 16))
    return ssrc, sdst, msl, cb


def _tc_prep1(xp_ref, W1_ref, as1_ref, ad1_ref, le1_ref, ae1_ref,
              pA_ref, pB_ref, pC_ref,
              h_ref, ssrc_ref, sdst_ref, msl_ref, cb_ref, l0_ref, l1_ref):
    xp = xp_ref[...]
    h = jax.lax.dot(xp, W1_ref[...], preferred_element_type=f32)
    pA = pA_ref[...]
    pB = pB_ref[...]
    pC = pC_ref[...]
    cnt = jnp.maximum(pC[:NPAD] + pC[NPAD:], 1.0)
    loop0 = (pA[:NPAD] + pA[NPAD:]) / cnt
    loop1 = (pB[:NPAD] + pB[NPAD:]) / cnt
    ssrc, sdst, msl, cb = _attn_scalars(
        h, as1_ref[...], ad1_ref[...], le1_ref[...], ae1_ref[...], loop0, loop1)
    h_ref[...] = h
    ssrc_ref[...] = ssrc
    sdst_ref[...] = sdst
    msl_ref[...] = msl
    cb_ref[...] = cb
    l0_ref[...] = loop0
    l1_ref[...] = loop1


def _tc_fin(outp_ref, denp_ref, h_ref, b_ref, xa_ref):
    denp = denp_ref[...]
    den = denp[:NPAD] + denp[NPAD:] + 1.0
    xa = (outp_ref[0] + outp_ref[1] + h_ref[...]) / den[:, None]
    xa = jnp.maximum(xa + b_ref[...], 0.0)
    rmask = lax.broadcasted_iota(i32, (NPAD, H), 0) < N
    xa_ref[...] = jnp.where(rmask, xa, 0.0)


def _tc_prepn(xa_ref, xp_ref, W_ref, as_ref, ad_ref, le_ref, ae_ref,
              l0_ref, l1_ref,
              hn_ref, ssrc_ref, sdst_ref, msl_ref, cb_ref):
    hn = (jax.lax.dot(xa_ref[...], W_ref[0:H, :], preferred_element_type=f32)
          + jax.lax.dot(xp_ref[...], W_ref[H:, :], preferred_element_type=f32))
    ssrc, sdst, msl, cb = _attn_scalars(
        hn, as_ref[...], ad_ref[...], le_ref[...], ae_ref[...],
        l0_ref[...], l1_ref[...])
    hn_ref[...] = hn
    ssrc_ref[...] = ssrc
    sdst_ref[...] = sdst
    msl_ref[...] = msl
    cb_ref[...] = cb


def _tc_prep4(xa_ref, xp_ref, W_ref, as_ref, ad_ref, le_ref, ae_ref,
              l0_ref, l1_ref,
              hn_ref, ssrc_ref, sdst_ref, msl_ref, cb_ref):
    v1 = W_ref[0:H, 0]
    v2 = W_ref[H:, 0]
    hn = jnp.sum(xa_ref[...] * v1, axis=-1) + jnp.sum(xp_ref[...] * v2, axis=-1)
    ssrc = hn * as_ref[0]
    sdst = hn * ad_ref[0]
    cvec = le_ref[:, 0] * ae_ref[0]
    t = ssrc + sdst + l0_ref[...] * cvec[0] + l1_ref[...] * cvec[1]
    msl = jnp.maximum(t, t * 0.2)
    hn_ref[...] = hn
    ssrc_ref[...] = ssrc
    sdst_ref[...] = sdst
    msl_ref[...] = msl
    cb_ref[...] = jnp.broadcast_to(cvec[:, None], (2, 16))


def _tc_head(outp4_ref, denp4_ref, h4f_ref, b4_ref, xa3_ref,
             Wv1_ref, bv1_ref, Wv2_ref, bv2_ref,
             px_ref, vx_ref):
    denp4 = denp4_ref[...]
    outp4 = outp4_ref[...]
    den = denp4[:NPAD] + denp4[NPAD:] + 1.0
    p = (outp4[:NPAD] + outp4[NPAD:] + h4f_ref[...]) / den + b4_ref[0]
    p = jnp.maximum(p, 0.0)
    px_ref[...] = p[:N][None, :]
    v = jnp.sum(xa3_ref[...], axis=0) / float(N)
    vx = jnp.maximum(v @ Wv1_ref[...] + bv1_ref[...], 0.0)
    vx_ref[...] = (vx @ Wv2_ref[...] + bv2_ref[...])[None, :]


def _pc(fn, out_shape):
    return pl.pallas_call(fn, out_shape=out_shape)


def kernel(x, edge_index, edge_attr, W1, as1, ad1, ae1, le1, b1,
           W2, as2, ad2, ae2, le2, b2, W3, as3, ad3, ae3, le3, b3,
           W4, as4, ad4, ae4, le4, b4, Wv1, bv1, Wv2, bv2):
    src = edge_index[0]
    dst = edge_index[1]
    pad_e = EPAD - E
    padi = jnp.full((pad_e,), N, i32)
    src2 = jnp.concatenate([src, padi]).reshape(ER, 128)
    dst2 = jnp.concatenate([dst, padi]).reshape(ER, 128)
    padf = jnp.zeros((pad_e,), f32)
    ew0 = jnp.concatenate([edge_attr[:, 0], padf]).reshape(ER, 128)
    ew1 = jnp.concatenate([edge_attr[:, 1], padf]).reshape(ER, 128)
    xp = jnp.pad(x, ((0, NPAD - N), (0, 0)))
    zN = jnp.zeros((NPAD,), f32)
    zNH = jnp.zeros((NPAD, H), f32)
    onesR = jnp.ones((RPT, 128), f32)

    pA, pB, pC = _sc_pass0(dst2, ew0, ew1, onesR, zN)

    sds = jax.ShapeDtypeStruct
    h1, ssrc, sdst, msl, cb, loop0, loop1 = _pc(
        _tc_prep1,
        (sds((NPAD, H), f32), sds((NPAD,), f32), sds((NPAD,), f32),
         sds((NPAD,), f32), sds((2, 16), f32), sds((NPAD,), f32),
         sds((NPAD,), f32)),
    )(xp, W1, as1, ad1, le1, ae1, pA, pB, pC)

    layers = [
        (b1, W2, as2, ad2, le2, ae2),
        (b2, W3, as3, ad3, le3, ae3),
        (b3, W4, as4, ad4, le4, ae4),
    ]

    h_cur = h1
    xa3 = None
    h4flat = None
    for li, (b_l, Wn, asn, adn, len_, aen) in enumerate(layers):
        denp, outp = _sc_layer(src2, dst2, ew0, ew1, ssrc, sdst, msl, cb,
                               h_cur, zN, zNH)
        xa = _pc(_tc_fin, sds((NPAD, H), f32))(outp, denp, h_cur, b_l)
        if li < 2:
            h_cur, ssrc, sdst, msl, cb = _pc(
                _tc_prepn,
                (sds((NPAD, H), f32), sds((NPAD,), f32), sds((NPAD,), f32),
                 sds((NPAD,), f32), sds((2, 16), f32)),
            )(xa, xp, Wn, asn, adn, len_, aen, loop0, loop1)
        else:
            h4flat, ssrc, sdst, msl, cb = _pc(
                _tc_prep4,
                (sds((NPAD,), f32), sds((NPAD,), f32), sds((NPAD,), f32),
                 sds((NPAD,), f32), sds((2, 16), f32)),
            )(xa, xp, Wn, asn, adn, len_, aen, loop0, loop1)
            xa3 = xa

    denp4, outp4 = _sc_layer4(src2, dst2, ew0, ew1, ssrc, sdst, msl, cb,
                              h4flat, zN)

    px, vx = _pc(
        _tc_head,
        (sds((1, N), f32), sds((1, 1), f32)),
    )(outp4, denp4, h4flat, b4, xa3, Wv1, bv1, Wv2, bv2)
    return (px, vx)


# speedup vs baseline: 103.7775x; 1.3444x over previous
import functools

import jax
import jax.numpy as jnp
from jax import lax
from jax.experimental import pallas as pl
from jax.experimental.pallas import tpu as pltpu
from jax.experimental.pallas import tpu_sc as plsc

N = 10000
NPAD = 10112
E = 320000
NW = 32
RPT = 80
EPW = RPT * 128
EPAD = NW * EPW
ER = EPAD // 128
H = 16
GP = 8
NG = RPT // GP

_mesh = plsc.VectorSubcoreMesh(core_axis_name="c", subcore_axis_name="s")
_sc_params = pltpu.CompilerParams(needs_layout_passes=False,
                                  use_tc_tiling_on_sc=False)
f32 = jnp.float32
i32 = jnp.int32


def _wid_base():
    c = lax.axis_index("c")
    s = lax.axis_index("s")
    return c, s, (c * 16 + s) * RPT


@functools.partial(
    pl.kernel,
    out_type=(jax.ShapeDtypeStruct((2 * NPAD,), f32),
              jax.ShapeDtypeStruct((2 * NPAD,), f32),
              jax.ShapeDtypeStruct((2 * NPAD,), f32)),
    mesh=_mesh,
    compiler_params=_sc_params,
    scratch_types=[
        pltpu.VMEM((RPT, 128), i32),
        pltpu.VMEM((RPT, 128), f32),
        pltpu.VMEM((RPT, 128), f32),
        pltpu.VMEM((RPT, 128), f32),
        pltpu.VMEM_SHARED((NPAD,), f32),
        pltpu.VMEM_SHARED((NPAD,), f32),
        pltpu.VMEM_SHARED((NPAD,), f32),
        pltpu.SemaphoreType.DMA,
    ],
)
def _sc_pass0(dst_h, ew0_h, ew1_h, ones_h, zeros_h, outA_h, outB_h, outC_h,
              dstb, e0b, e1b, oneb, accA, accB, accC, ssem):
    c, s, base = _wid_base()

    @pl.when(s == 0)
    def _():
        pltpu.sync_copy(zeros_h, accA)
        pltpu.sync_copy(zeros_h, accB)
        pltpu.sync_copy(zeros_h, accC)

    pltpu.sync_copy(dst_h.at[pl.ds(base, RPT)], dstb)
    pltpu.sync_copy(ew0_h.at[pl.ds(base, RPT)], e0b)
    pltpu.sync_copy(ew1_h.at[pl.ds(base, RPT)], e1b)
    pltpu.sync_copy(ones_h, oneb)

    plsc.subcore_barrier()

    hs = {}
    for g in range(NG):
        if g >= 2:
            for hd in hs[g - 2]:
                hd.wait()
        hs[g] = []
        for b in range(GP):
            j = g * GP + b
            hs[g].append(pltpu.async_copy(e0b.at[j], accA.at[dstb.at[j]],
                                          ssem, add=True))
            hs[g].append(pltpu.async_copy(e1b.at[j], accB.at[dstb.at[j]],
                                          ssem, add=True))
            hs[g].append(pltpu.async_copy(oneb.at[j], accC.at[dstb.at[j]],
                                          ssem, add=True))
    for g in (NG - 2, NG - 1):
        for hd in hs[g]:
            hd.wait()
    plsc.subcore_barrier()

    @pl.when(s == 0)
    def _():
        pltpu.sync_copy(accA, outA_h.at[pl.ds(c * NPAD, NPAD)])
        pltpu.sync_copy(accB, outB_h.at[pl.ds(c * NPAD, NPAD)])
        pltpu.sync_copy(accC, outC_h.at[pl.ds(c * NPAD, NPAD)])


@functools.partial(
    pl.kernel,
    out_type=(jax.ShapeDtypeStruct((2 * NPAD,), f32),
              jax.ShapeDtypeStruct((2, NPAD, H), f32)),
    mesh=_mesh,
    compiler_params=_sc_params,
    scratch_types=[
        pltpu.VMEM((RPT, 128), i32),
        pltpu.VMEM((RPT, 128), i32),
        pltpu.VMEM((RPT, 128), f32),
        pltpu.VMEM((RPT, 128), f32),
        pltpu.VMEM((RPT, 128), f32),
        pltpu.VMEM((NPAD,), f32),
        pltpu.VMEM((NPAD,), f32),
        pltpu.VMEM((NPAD,), f32),
        pltpu.VMEM((2, 16), f32),
        pltpu.VMEM((2, GP, 128, H), f32),
        pltpu.VMEM_SHARED((NPAD,), f32),
        pltpu.VMEM_SHARED((NPAD, H), f32),
        pltpu.SemaphoreType.DMA,
        pltpu.SemaphoreType.DMA,
    ],
)
def _sc_layer(src_h, dst_h, ew0_h, ew1_h, ssrc_h, sdst_h, msl_h, cb_h, h_h,
              zeros_h, zeros2_h,
              denp_h, outp_h,
              srcb, dstb, e0b, e1b, exb, ssrcT, sdstT, mslT, cbb, rows8,
              den_sh, out_sh, gsem, ssem):
    c, s, base = _wid_base()

    @pl.when(s == 0)
    def _():
        pltpu.sync_copy(zeros_h, den_sh)
        pltpu.sync_copy(zeros2_h, out_sh)

    pltpu.sync_copy(src_h.at[pl.ds(base, RPT)], srcb)
    pltpu.sync_copy(dst_h.at[pl.ds(base, RPT)], dstb)
    pltpu.sync_copy(ew0_h.at[pl.ds(base, RPT)], e0b)
    pltpu.sync_copy(ew1_h.at[pl.ds(base, RPT)], e1b)
    pltpu.sync_copy(ssrc_h, ssrcT)
    pltpu.sync_copy(sdst_h, sdstT)
    pltpu.sync_copy(msl_h, mslT)
    pltpu.sync_copy(cb_h, cbb)

    c0 = cbb[0, :]
    c1 = cbb[1, :]

    @plsc.parallel_loop(0, RPT, 1, unroll=2)
    def _(j):
        for k in range(8):
            sl = pl.ds(k * 16, 16)
            s16 = srcb[j, sl]
            d16 = dstb[j, sl]
            e0 = e0b[j, sl]
            e1 = e1b[j, sl]
            g1 = plsc.load_gather(ssrcT, [s16])
            g2 = plsc.load_gather(sdstT, [d16])
            g3 = plsc.load_gather(mslT, [d16])
            al = g1 + g2 + e0 * c0 + e1 * c1
            al = jnp.maximum(al, al * 0.2)
            t = jnp.minimum(al - g3, 60.0)
            exb[j, sl] = jnp.exp(t)

    plsc.subcore_barrier()

    zi = jnp.zeros((16,), i32)
    gh = {}
    sh = {}
    dh = {}

    def scale_scatter(g):
        bb = g % 2
        sh[g] = []
        for b in range(GP):
            j = g * GP + b
            gh[g][b].wait()
            jsplat = zi + j

            @plsc.parallel_loop(0, 128, 1, unroll=4)
            def _(i):
                gsc = plsc.load_gather(exb, [jsplat, zi + i])
                rows8[bb, b, i, :] = rows8[bb, b, i, :] * gsc

            sh[g].append(pltpu.async_copy(rows8.at[bb, b],
                                          out_sh.at[dstb.at[j]], ssem,
                                          add=True))

    for g in range(NG):
        if g >= 2:
            for hd in sh[g - 2]:
                hd.wait()
            for hd in dh[g - 2]:
                hd.wait()
        bb = g % 2
        dh[g] = []
        gh[g] = []
        for b in range(GP):
            j = g * GP + b
            dh[g].append(pltpu.async_copy(exb.at[j], den_sh.at[dstb.at[j]],
                                          ssem, add=True))
            gh[g].append(pltpu.async_copy(h_h.at[srcb.at[j]],
                                          rows8.at[bb, b], gsem))
        if g >= 1:
            scale_scatter(g - 1)
    scale_scatter(NG - 1)
    for g in (NG - 2, NG - 1):
        for hd in sh[g]:
            hd.wait()
        for hd in dh[g]:
            hd.wait()
    plsc.subcore_barrier()

    @pl.when(s == 0)
    def _():
        pltpu.sync_copy(den_sh, denp_h.at[pl.ds(c * NPAD, NPAD)])
        pltpu.sync_copy(out_sh, outp_h.at[c])


@functools.partial(
    pl.kernel,
    out_type=(jax.ShapeDtypeStruct((2 * NPAD,), f32),
              jax.ShapeDtypeStruct((2 * NPAD,), f32)),
    mesh=_mesh,
    compiler_params=_sc_params,
    scratch_types=[
        pltpu.VMEM((RPT, 128), i32),
        pltpu.VMEM((RPT, 128), i32),
        pltpu.VMEM((RPT, 128), f32),
        pltpu.VMEM((RPT, 128), f32),
        pltpu.VMEM((RPT, 128), f32),
        pltpu.VMEM((RPT, 128), f32),
        pltpu.VMEM((NPAD,), f32),
        pltpu.VMEM((NPAD,), f32),
        pltpu.VMEM((NPAD,), f32),
        pltpu.VMEM((NPAD,), f32),
        pltpu.VMEM((2, 16), f32),
        pltpu.VMEM_SHARED((NPAD,), f32),
        pltpu.VMEM_SHARED((NPAD,), f32),
        pltpu.SemaphoreType.DMA,
    ],
)
def _sc_layer4(src_h, dst_h, ew0_h, ew1_h, ssrc_h, sdst_h, msl_h, cb_h, h1_h,
               zeros_h,
               denp_h, outp_h,
               srcb, dstb, e0b, e1b, exb, vb, ssrcT, sdstT, mslT, h1T, cbb,
               den_sh, out_sh, ssem):
    c, s, base = _wid_base()

    @pl.when(s == 0)
    def _():
        pltpu.sync_copy(zeros_h, den_sh)
        pltpu.sync_copy(zeros_h, out_sh)

    pltpu.sync_copy(src_h.at[pl.ds(base, RPT)], srcb)
    pltpu.sync_copy(dst_h.at[pl.ds(base, RPT)], dstb)
    pltpu.sync_copy(ew0_h.at[pl.ds(base, RPT)], e0b)
    pltpu.sync_copy(ew1_h.at[pl.ds(base, RPT)], e1b)
    pltpu.sync_copy(ssrc_h, ssrcT)
    pltpu.sync_copy(sdst_h, sdstT)
    pltpu.sync_copy(msl_h, mslT)
    pltpu.sync_copy(h1_h, h1T)
    pltpu.sync_copy(cb_h, cbb)

    c0 = cbb[0, :]
    c1 = cbb[1, :]

    @plsc.parallel_loop(0, RPT, 1, unroll=2)
    def _(j):
        for k in range(8):
            sl = pl.ds(k * 16, 16)
            s16 = srcb[j, sl]
            d16 = dstb[j, sl]
            e0 = e0b[j, sl]
            e1 = e1b[j, sl]
            g1 = plsc.load_gather(ssrcT, [s16])
            g2 = plsc.load_gather(sdstT, [d16])
            g3 = plsc.load_gather(mslT, [d16])
            hg = plsc.load_gather(h1T, [s16])
            al = g1 + g2 + e0 * c0 + e1 * c1
            al = jnp.maximum(al, al * 0.2)
            t = jnp.minimum(al - g3, 60.0)
            ex16 = jnp.exp(t)
            exb[j, sl] = ex16
            vb[j, sl] = ex16 * hg

    plsc.subcore_barrier()

    hs = {}
    for g in range(NG):
        if g >= 2:
            for hd in hs[g - 2]:
                hd.wait()
        hs[g] = []
        for b in range(GP):
            j = g * GP + b
            hs[g].append(pltpu.async_copy(exb.at[j], den_sh.at[dstb.at[j]],
                                          ssem, add=True))
            hs[g].append(pltpu.async_copy(vb.at[j], out_sh.at[dstb.at[j]],
                                          ssem, add=True))
    for g in (NG - 2, NG - 1):
        for hd in hs[g]:
            hd.wait()
    plsc.subcore_barrier()

    @pl.when(s == 0)
    def _():
        pltpu.sync_copy(den_sh, denp_h.at[pl.ds(c * NPAD, NPAD)])
        pltpu.sync_copy(out_sh, outp_h.at[pl.ds(c * NPAD, NPAD)])


def _attn_scalars(h, a_s, a_d, le, ae, loop0, loop1):
    ssrc = jnp.sum(h * a_s, axis=-1)
    sdst = jnp.sum(h * a_d, axis=-1)
    cvec = le @ ae
    t = ssrc + sdst + loop0 * cvec[0] + loop1 * cvec[1]
    msl = jnp.maximum(t, t * 0.2)
    cb = jnp.broadcast_to(cvec[:, None], (2, 16))
    return ssrc, sdst, msl, cb


def _tc_prep1(xp_ref, W1_ref, as1_ref, ad1_ref, le1_ref, ae1_ref,
              pA_ref, pB_ref, pC_ref,
              h_ref, ssrc_ref, sdst_ref, msl_ref, cb_ref, l0_ref, l1_ref):
    xp = xp_ref[...]
    h = jax.lax.dot(xp, W1_ref[...], preferred_element_type=f32)
    pA = pA_ref[...]
    pB = pB_ref[...]
    pC = pC_ref[...]
    cnt = jnp.maximum(pC[:NPAD] + pC[NPAD:], 1.0)
    loop0 = (pA[:NPAD] + pA[NPAD:]) / cnt
    loop1 = (pB[:NPAD] + pB[NPAD:]) / cnt
    ssrc, sdst, msl, cb = _attn_scalars(
        h, as1_ref[...], ad1_ref[...], le1_ref[...], ae1_ref[...], loop0, loop1)
    h_ref[...] = h
    ssrc_ref[...] = ssrc
    sdst_ref[...] = sdst
    msl_ref[...] = msl
    cb_ref[...] = cb
    l0_ref[...] = loop0
    l1_ref[...] = loop1


def _tc_fin(outp_ref, denp_ref, h_ref, b_ref, xa_ref):
    denp = denp_ref[...]
    den = denp[:NPAD] + denp[NPAD:] + 1.0
    xa = (outp_ref[0] + outp_ref[1] + h_ref[...]) / den[:, None]
    xa = jnp.maximum(xa + b_ref[...], 0.0)
    rmask = lax.broadcasted_iota(i32, (NPAD, H), 0) < N
    xa_ref[...] = jnp.where(rmask, xa, 0.0)


def _tc_prepn(xa_ref, xp_ref, W_ref, as_ref, ad_ref, le_ref, ae_ref,
              l0_ref, l1_ref,
              hn_ref, ssrc_ref, sdst_ref, msl_ref, cb_ref):
    hn = (jax.lax.dot(xa_ref[...], W_ref[0:H, :], preferred_element_type=f32)
          + jax.lax.dot(xp_ref[...], W_ref[H:, :], preferred_element_type=f32))
    ssrc, sdst, msl, cb = _attn_scalars(
        hn, as_ref[...], ad_ref[...], le_ref[...], ae_ref[...],
        l0_ref[...], l1_ref[...])
    hn_ref[...] = hn
    ssrc_ref[...] = ssrc
    sdst_ref[...] = sdst
    msl_ref[...] = msl
    cb_ref[...] = cb


def _tc_prep4(xa_ref, xp_ref, W_ref, as_ref, ad_ref, le_ref, ae_ref,
              l0_ref, l1_ref,
              hn_ref, ssrc_ref, sdst_ref, msl_ref, cb_ref):
    v1 = W_ref[0:H, 0]
    v2 = W_ref[H:, 0]
    hn = jnp.sum(xa_ref[...] * v1, axis=-1) + jnp.sum(xp_ref[...] * v2, axis=-1)
    ssrc = hn * as_ref[0]
    sdst = hn * ad_ref[0]
    cvec = le_ref[:, 0] * ae_ref[0]
    t = ssrc + sdst + l0_ref[...] * cvec[0] + l1_ref[...] * cvec[1]
    msl = jnp.maximum(t, t * 0.2)
    hn_ref[...] = hn
    ssrc_ref[...] = ssrc
    sdst_ref[...] = sdst
    msl_ref[...] = msl
    cb_ref[...] = jnp.broadcast_to(cvec[:, None], (2, 16))


def _tc_head(outp4_ref, denp4_ref, h4f_ref, b4_ref, xa3_ref,
             Wv1_ref, bv1_ref, Wv2_ref, bv2_ref,
             px_ref, vx_ref):
    denp4 = denp4_ref[...]
    outp4 = outp4_ref[...]
    den = denp4[:NPAD] + denp4[NPAD:] + 1.0
    p = (outp4[:NPAD] + outp4[NPAD:] + h4f_ref[...]) / den + b4_ref[0]
    p = jnp.maximum(p, 0.0)
    px_ref[...] = p[:N][None, :]
    v = jnp.sum(xa3_ref[...], axis=0) / float(N)
    vx = jnp.maximum(v @ Wv1_ref[...] + bv1_ref[...], 0.0)
    vx_ref[...] = (vx @ Wv2_ref[...] + bv2_ref[...])[None, :]


def _pc(fn, out_shape):
    return pl.pallas_call(fn, out_shape=out_shape)


def kernel(x, edge_index, edge_attr, W1, as1, ad1, ae1, le1, b1,
           W2, as2, ad2, ae2, le2, b2, W3, as3, ad3, ae3, le3, b3,
           W4, as4, ad4, ae4, le4, b4, Wv1, bv1, Wv2, bv2):
    src = edge_index[0]
    dst = edge_index[1]
    pad_e = EPAD - E
    padi = N + (jnp.arange(pad_e, dtype=i32) % (NPAD - N))
    src2 = jnp.concatenate([src, padi]).reshape(ER, 128)
    dst2 = jnp.concatenate([dst, padi]).reshape(ER, 128)
    padf = jnp.zeros((pad_e,), f32)
    ew0 = jnp.concatenate([edge_attr[:, 0], padf]).reshape(ER, 128)
    ew1 = jnp.concatenate([edge_attr[:, 1], padf]).reshape(ER, 128)
    xp = jnp.pad(x, ((0, NPAD - N), (0, 0)))
    zN = jnp.zeros((NPAD,), f32)
    zNH = jnp.zeros((NPAD, H), f32)
    onesR = jnp.ones((RPT, 128), f32)

    pA, pB, pC = _sc_pass0(dst2, ew0, ew1, onesR, zN)

    sds = jax.ShapeDtypeStruct
    h1, ssrc, sdst, msl, cb, loop0, loop1 = _pc(
        _tc_prep1,
        (sds((NPAD, H), f32), sds((NPAD,), f32), sds((NPAD,), f32),
         sds((NPAD,), f32), sds((2, 16), f32), sds((NPAD,), f32),
         sds((NPAD,), f32)),
    )(xp, W1, as1, ad1, le1, ae1, pA, pB, pC)

    layers = [
        (b1, W2, as2, ad2, le2, ae2),
        (b2, W3, as3, ad3, le3, ae3),
        (b3, W4, as4, ad4, le4, ae4),
    ]

    h_cur = h1
    xa3 = None
    h4flat = None
    for li, (b_l, Wn, asn, adn, len_, aen) in enumerate(layers):
        denp, outp = _sc_layer(src2, dst2, ew0, ew1, ssrc, sdst, msl, cb,
                               h_cur, zN, zNH)
        xa = _pc(_tc_fin, sds((NPAD, H), f32))(outp, denp, h_cur, b_l)
        if li < 2:
            h_cur, ssrc, sdst, msl, cb = _pc(
                _tc_prepn,
                (sds((NPAD, H), f32), sds((NPAD,), f32), sds((NPAD,), f32),
                 sds((NPAD,), f32), sds((2, 16), f32)),
            )(xa, xp, Wn, asn, adn, len_, aen, loop0, loop1)
        else:
            h4flat, ssrc, sdst, msl, cb = _pc(
                _tc_prep4,
                (sds((NPAD,), f32), sds((NPAD,), f32), sds((NPAD,), f32),
                 sds((NPAD,), f32), sds((2, 16), f32)),
            )(xa, xp, Wn, asn, adn, len_, aen, loop0, loop1)
            xa3 = xa

    denp4, outp4 = _sc_layer4(src2, dst2, ew0, ew1, ssrc, sdst, msl, cb,
                              h4flat, zN)

    px, vx = _pc(
        _tc_head,
        (sds((1, N), f32), sds((1, 1), f32)),
    )(outp4, denp4, h4flat, b4, xa3, Wv1, bv1, Wv2, bv2)
    return (px, vx)
